# Initial kernel scaffold; baseline (speedup 1.0000x reference)
import jax
import jax.numpy as jnp
from jax.experimental import pallas as pl


def kernel(entity_emb, user_emb, user_emb_cf, item_emb_cf, relation_weight, W1_w, W1_b, W2_w, W2_b, edge_index, edge_type, interact_mat):
    raise NotImplementedError("write your pallas kernel here")


# speedup vs baseline: 4.1816x; 4.1816x over previous
import dataclasses
import functools

import jax
import jax.numpy as jnp
from jax import lax
from jax.experimental import pallas as pl
from jax.experimental.pallas import tpu as pltpu
from jax.experimental.pallas import tpu_sc as plsc

N_USERS = 30000
N_ITEMS = 20000
N_ENTITIES = 50000
N_RELATIONS = 16
N_EDGES = 800000
N_INTER = 500000
DIM = 64
GAMMA = 0.6
MAX_ITER = 2

NHIST = N_ENTITIES * 32
EPAD = 16 * 512 * 98
MPAD = 16 * 512 * 62
SROWS = 51200
UROWS = 30720
IROWS = 20480
NREP = 256

f32 = jnp.float32
i32 = jnp.int32

_MESH = plsc.VectorSubcoreMesh(core_axis_name="c", subcore_axis_name="s")

_CP = pltpu.CompilerParams()
if "needs_layout_passes" in pltpu.CompilerParams.__dataclass_fields__:
    _CP = dataclasses.replace(_CP, needs_layout_passes=False)
if "use_tc_tiling_on_sc" in pltpu.CompilerParams.__dataclass_fields__:
    _CP = dataclasses.replace(_CP, use_tc_tiling_on_sc=False)


def _iota16():
    return lax.iota(i32, 16)


def _zero_fill(ref, n):
    @pl.loop(0, n, step=16)
    def _(i):
        ref[pl.ds(i, 16)] = jnp.zeros((16,), f32)


def _zero_fill2(ref, rows, width=16):
    @pl.loop(0, rows)
    def _(r):
        for cc in range(0, width, 16):
            ref[r, pl.ds(cc, 16)] = jnp.zeros((16,), f32)


def _hist_body(head_hbm, tail_hbm, type_hbm, out_hbm,
               h_ref, t_ref, tp_ref, idx_ref, val_ref, zbuf, acc):
    cid = lax.axis_index("c")
    sid = lax.axis_index("s")
    nchunk = NHIST // 6400

    _zero_fill(zbuf, 6400)

    @pl.loop(0, 16)
    def _(it):
        ci = sid + it * 16

        @pl.when(ci < nchunk)
        def _():
            pltpu.sync_copy(zbuf, acc.at[pl.ds(ci * 6400, 6400)])

    plsc.subcore_barrier()

    wid = sid * 2 + cid
    nb = EPAD // (32 * 512)

    @pl.loop(0, nb)
    def _(b):
        base = (wid * nb + b) * 512
        pltpu.sync_copy(head_hbm.at[pl.ds(base, 512)], h_ref)
        pltpu.sync_copy(tail_hbm.at[pl.ds(base, 512)], t_ref)
        pltpu.sync_copy(type_hbm.at[pl.ds(base, 512)], tp_ref)

        @pl.loop(0, 32)
        def _(k):
            h = h_ref[pl.ds(k * 16, 16)]
            t = t_ref[pl.ds(k * 16, 16)]
            tp = tp_ref[pl.ds(k * 16, 16)]
            ha = (h < N_ITEMS).astype(i32)
            ta = (t < N_ITEMS).astype(i32)
            same_i = 1 - (ha ^ ta)
            flat = h * 32 + same_i * 16 + tp
            gi = base + k * 16 + _iota16()
            val = (gi < N_EDGES).astype(f32)
            idx_ref[k // 8, pl.ds((k % 8) * 16, 16)] = flat
            val_ref[k // 8, pl.ds((k % 8) * 16, 16)] = val

        @pl.loop(0, 4)
        def _(j):
            pltpu.sync_copy(val_ref.at[j], acc.at[idx_ref.at[j]], add=True)

    plsc.subcore_barrier()

    @pl.loop(0, 16)
    def _(it):
        ci = sid + it * 16

        @pl.when(ci < nchunk)
        def _():
            pltpu.sync_copy(acc.at[pl.ds(ci * 6400, 6400)], zbuf)
            pltpu.sync_copy(zbuf, out_hbm.at[cid].at[pl.ds(ci * 6400, 6400)])


@jax.jit
def _hist_call(head, tail, etype):
    k = pl.kernel(
        _hist_body,
        out_type=jax.ShapeDtypeStruct((2, NHIST), f32),
        mesh=_MESH,
        compiler_params=_CP,
        scratch_types=[
            pltpu.VMEM((512,), i32),
            pltpu.VMEM((512,), i32),
            pltpu.VMEM((512,), i32),
            pltpu.VMEM((4, 128), i32),
            pltpu.VMEM((4, 128), f32),
            pltpu.VMEM((6400,), f32),
            pltpu.VMEM_SHARED((NHIST,), f32),
        ],
    )
    return k(head, tail, etype)


def _scatsum_body(head_hbm, tail_hbm, type_hbm, entq_hbm, mq_hbm, out_hbm,
                  h_ref, t_ref, tp_ref, gidx, midx, didx,
                  ent_rows, mult_rows, zrow, acc):
    cid = lax.axis_index("c")
    sid = lax.axis_index("s")
    nb = EPAD // (16 * 512)

    @pl.loop(0, 4)
    def _(j):
        _zero_fill2(zrow, 1600)

        @pl.loop(0, 2)
        def _(q):
            pltpu.sync_copy(zrow, acc.at[pl.ds(sid * 3200 + q * 1600, 1600)])

        plsc.subcore_barrier()

        @pl.loop(0, nb)
        def _(b):
            base = (sid * nb + b) * 512
            pltpu.sync_copy(head_hbm.at[pl.ds(base, 512)], h_ref)
            pltpu.sync_copy(tail_hbm.at[pl.ds(base, 512)], t_ref)
            pltpu.sync_copy(type_hbm.at[pl.ds(base, 512)], tp_ref)

            @pl.loop(0, 32)
            def _(k):
                h = h_ref[pl.ds(k * 16, 16)]
                t = t_ref[pl.ds(k * 16, 16)]
                tp = tp_ref[pl.ds(k * 16, 16)]
                ha = (h < N_ITEMS).astype(i32)
                ta = (t < N_ITEMS).astype(i32)
                same_i = 1 - (ha ^ ta)
                gi = base + k * 16 + _iota16()
                valid = (gi < N_EDGES).astype(i32)
                mine = valid * (1 - (same_i ^ cid))
                jit = gi & (NREP - 1)
                kr = k // 8
                kc = (k % 8) * 16
                gidx[kr, pl.ds(kc, 16)] = t * 4 + j
                midx[kr, pl.ds(kc, 16)] = (tp * NREP + jit) * 4 + j
                didx[kr, pl.ds(kc, 16)] = (mine * h
                                           + (1 - mine) * (N_ENTITIES + _iota16()))

            @pl.loop(0, 4)
            def _(q):
                pltpu.sync_copy(entq_hbm.at[gidx.at[q]],
                                ent_rows.at[pl.ds(q * 128, 128)])

            @pl.when(cid == 0)
            def _():
                @pl.loop(0, 4)
                def _(q):
                    pltpu.sync_copy(mq_hbm.at[midx.at[q]],
                                    mult_rows.at[pl.ds(q * 128, 128)])

                @pl.loop(0, 512)
                def _(r):
                    ent_rows[r, :] = ent_rows[r, :] * mult_rows[r, :]

            @pl.loop(0, 4)
            def _(q):
                pltpu.sync_copy(ent_rows.at[pl.ds(q * 128, 128)],
                                acc.at[didx.at[q]], add=True)

        plsc.subcore_barrier()

        @pl.loop(0, 2)
        def _(q):
            off = sid * 3200 + q * 1600
            pltpu.sync_copy(acc.at[pl.ds(off, 1600)], zrow)
            pltpu.sync_copy(
                zrow, out_hbm.at[pl.ds((cid * 4 + j) * SROWS + off, 1600)])

        plsc.subcore_barrier()


@jax.jit
def _scatsum_call(head, tail, etype, entq, mq):
    k = pl.kernel(
        _scatsum_body,
        out_type=jax.ShapeDtypeStruct((2 * 4 * SROWS, 16), f32),
        mesh=_MESH,
        compiler_params=_CP,
        scratch_types=[
            pltpu.VMEM((512,), i32),
            pltpu.VMEM((512,), i32),
            pltpu.VMEM((512,), i32),
            pltpu.VMEM((4, 128), i32),
            pltpu.VMEM((4, 128), i32),
            pltpu.VMEM((4, 128), i32),
            pltpu.VMEM((512, 16), f32),
            pltpu.VMEM((512, 16), f32),
            pltpu.VMEM((1600, 16), f32),
            pltpu.VMEM_SHARED((SROWS, 16), f32),
        ],
    )
    return k(head, tail, etype, entq, mq)


def _dots_body(rowp_hbm, colp_hbm, u2_hbm, z2_hbm, e_out, d_out,
               row_ref, col_ref, uidx, zidx, didx,
               u_rows, z_rows, dots, e_lin, zd, acc_d):
    cid = lax.axis_index("c")
    sid = lax.axis_index("s")

    _zero_fill(zd, 1920)
    pltpu.sync_copy(zd, acc_d.at[pl.ds(sid * 1920, 1920)])
    plsc.subcore_barrier()

    nb = MPAD // (16 * 512)

    @pl.loop(0, nb)
    def _(b):
        base = (sid * nb + b) * 512
        pltpu.sync_copy(rowp_hbm.at[pl.ds(base, 512)], row_ref)
        pltpu.sync_copy(colp_hbm.at[pl.ds(base, 512)], col_ref)

        @pl.loop(0, 32)
        def _(k):
            r = row_ref[pl.ds(k * 16, 16)]
            c = col_ref[pl.ds(k * 16, 16)]
            kr = k // 8
            kc = (k % 8) * 16
            uidx[kr, pl.ds(kc, 16)] = cid * N_USERS + r
            zidx[kr, pl.ds(kc, 16)] = cid * N_ITEMS + c
            didx[kr, pl.ds(kc, 16)] = r

        @pl.loop(0, 4)
        def _(q):
            pltpu.sync_copy(u2_hbm.at[uidx.at[q]],
                            u_rows.at[pl.ds(q * 128, 128)])
            pltpu.sync_copy(z2_hbm.at[zidx.at[q]],
                            z_rows.at[pl.ds(q * 128, 128)])

        @pl.loop(0, 32)
        def _(g):
            @pl.loop(0, 16)
            def _(r16):
                rr = g * 16 + r16
                part = (u_rows[rr, pl.ds(0, 16)] * z_rows[rr, pl.ds(0, 16)]
                        + u_rows[rr, pl.ds(16, 16)] * z_rows[rr, pl.ds(16, 16)]
                        + u_rows[rr, pl.ds(32, 16)] * z_rows[rr, pl.ds(32, 16)]
                        + u_rows[rr, pl.ds(48, 16)] * z_rows[rr, pl.ds(48, 16)])
                dots[pl.ds(r16 * 16, 16)] = part

            dv = plsc.load_gather(dots, [_iota16() * 16])
            for cc in range(1, 16):
                dv = dv + plsc.load_gather(dots, [_iota16() * 16 + cc])
            sig = 1.0 / (1.0 + jnp.exp(-dv))
            ev = jnp.exp(sig)
            gi = base + g * 16 + _iota16()
            ev = ev * (gi < N_INTER).astype(f32)
            e_lin[pl.ds(g * 16, 16)] = ev

        pltpu.sync_copy(e_lin, e_out.at[cid].at[pl.ds(base, 512)])

        @pl.loop(0, 4)
        def _(q):
            pltpu.sync_copy(e_lin.at[pl.ds(q * 128, 128)],
                            acc_d.at[didx.at[q]], add=True)

    plsc.subcore_barrier()
    pltpu.sync_copy(acc_d.at[pl.ds(sid * 1920, 1920)], zd)
    pltpu.sync_copy(zd, d_out.at[cid].at[pl.ds(sid * 1920, 1920)])


@jax.jit
def _dots_call(rowp, colp, u2, z2):
    k = pl.kernel(
        _dots_body,
        out_type=(jax.ShapeDtypeStruct((2, MPAD), f32),
                  jax.ShapeDtypeStruct((2, UROWS), f32)),
        mesh=_MESH,
        compiler_params=_CP,
        scratch_types=[
            pltpu.VMEM((512,), i32),
            pltpu.VMEM((512,), i32),
            pltpu.VMEM((4, 128), i32),
            pltpu.VMEM((4, 128), i32),
            pltpu.VMEM((4, 128), i32),
            pltpu.VMEM((512, 64), f32),
            pltpu.VMEM((512, 64), f32),
            pltpu.VMEM((256,), f32),
            pltpu.VMEM((512,), f32),
            pltpu.VMEM((1920,), f32),
            pltpu.VMEM_SHARED((UROWS,), f32),
        ],
    )
    return k(rowp, colp, u2, z2)


def _uacc_body(rowp_hbm, colp_hbm, e2_hbm, dflat_hbm, v2h_hbm,
               usum_out, mask_out,
               row_ref, col_ref, vidx, dpi, dci,
               ep_lin, ec_lin, dp_lin, dc_lin, s_lin, mask_lin,
               item_rows, zu, acc_u):
    cid = lax.axis_index("c")
    sid = lax.axis_index("s")

    nb = MPAD // (16 * 512)
    cidf = cid.astype(f32)

    @pl.loop(0, 2)
    def _(jj):
        _zero_fill2(zu, 384, 32)

        @pl.loop(0, 5)
        def _(q):
            pltpu.sync_copy(zu, acc_u.at[pl.ds(sid * 1920 + q * 384, 384)])

        plsc.subcore_barrier()

        @pl.loop(0, nb)
        def _(b):
            base = (sid * nb + b) * 512
            pltpu.sync_copy(rowp_hbm.at[pl.ds(base, 512)], row_ref)
            pltpu.sync_copy(colp_hbm.at[pl.ds(base, 512)], col_ref)
            pltpu.sync_copy(e2_hbm.at[0].at[pl.ds(base, 512)], ep_lin)
            pltpu.sync_copy(e2_hbm.at[1].at[pl.ds(base, 512)], ec_lin)

            @pl.loop(0, 32)
            def _(k):
                r = row_ref[pl.ds(k * 16, 16)]
                c = col_ref[pl.ds(k * 16, 16)]
                kr = k // 8
                kc = (k % 8) * 16
                vidx[kr, pl.ds(kc, 16)] = (cid * N_ITEMS + c) * 2 + jj
                dpi[kr, pl.ds(kc, 16)] = r
                dci[kr, pl.ds(kc, 16)] = UROWS + r

            @pl.loop(0, 4)
            def _(q):
                pltpu.sync_copy(dflat_hbm.at[dpi.at[q]],
                                dp_lin.at[pl.ds(q * 128, 128)])
                pltpu.sync_copy(dflat_hbm.at[dci.at[q]],
                                dc_lin.at[pl.ds(q * 128, 128)])
                pltpu.sync_copy(v2h_hbm.at[vidx.at[q]],
                                item_rows.at[pl.ds(q * 128, 128)])

            @pl.loop(0, 32)
            def _(g):
                sl = pl.ds(g * 16, 16)
                p = ep_lin[sl] / dp_lin[sl]
                pcf = ec_lin[sl] / dc_lin[sl]
                sigp = 1.0 / (1.0 + jnp.exp(-p))
                sigc = 1.0 / (1.0 + jnp.exp(-pcf))
                m = (jnp.abs(sigp - sigc) < GAMMA).astype(f32)
                gi = base + g * 16 + _iota16()
                validf = (gi < N_INTER).astype(f32)
                s = (p * (1.0 - cidf) + pcf * cidf) * m * validf
                s_lin[sl] = s
                mask_lin[sl] = m.astype(i32)

            @pl.loop(0, 512)
            def _(r):
                sv = plsc.load_gather(s_lin, [_iota16() * 0 + r])
                item_rows[r, pl.ds(0, 16)] = item_rows[r, pl.ds(0, 16)] * sv
                item_rows[r, pl.ds(16, 16)] = item_rows[r, pl.ds(16, 16)] * sv

            @pl.loop(0, 4)
            def _(q):
                pltpu.sync_copy(item_rows.at[pl.ds(q * 128, 128)],
                                acc_u.at[dpi.at[q]], add=True)

            @pl.when(cid + jj == 0)
            def _():
                pltpu.sync_copy(mask_lin, mask_out.at[pl.ds(base, 512)])

        plsc.subcore_barrier()

        @pl.loop(0, 5)
        def _(q):
            off = sid * 1920 + q * 384
            pltpu.sync_copy(acc_u.at[pl.ds(off, 384)], zu)
            pltpu.sync_copy(
                zu, usum_out.at[pl.ds((cid * 2 + jj) * UROWS + off, 384)])

        plsc.subcore_barrier()


@jax.jit
def _uacc_call(rowp, colp, e2, dflat, v2h):
    k = pl.kernel(
        _uacc_body,
        out_type=(jax.ShapeDtypeStruct((2 * 2 * UROWS, 32), f32),
                  jax.ShapeDtypeStruct((MPAD,), i32)),
        mesh=_MESH,
        compiler_params=_CP,
        scratch_types=[
            pltpu.VMEM((512,), i32),
            pltpu.VMEM((512,), i32),
            pltpu.VMEM((4, 128), i32),
            pltpu.VMEM((4, 128), i32),
            pltpu.VMEM((4, 128), i32),
            pltpu.VMEM((512,), f32),
            pltpu.VMEM((512,), f32),
            pltpu.VMEM((512,), f32),
            pltpu.VMEM((512,), f32),
            pltpu.VMEM((512,), f32),
            pltpu.VMEM((512,), i32),
            pltpu.VMEM((512, 32), f32),
            pltpu.VMEM((384, 32), f32),
            pltpu.VMEM_SHARED((UROWS, 32), f32),
        ],
    )
    return k(rowp, colp, e2, dflat, v2h)


def _iagg_body(rowp_hbm, colp_hbm, ucfh_hbm, isum_out, icnt_out,
               row_ref, col_ref, uidx, didx, cval,
               u_rows, zi, zc, acc_i, acc_c):
    cid = lax.axis_index("c")
    sid = lax.axis_index("s")

    _zero_fill(zc, 1280)
    pltpu.sync_copy(zc, acc_c.at[pl.ds(sid * 1280, 1280)])

    wid = sid * 2 + cid
    nb = MPAD // (32 * 512)

    @pl.loop(0, 2)
    def _(jj):
        _zero_fill2(zi, 320, 32)

        @pl.loop(0, 4)
        def _(q):
            pltpu.sync_copy(zi, acc_i.at[pl.ds(sid * 1280 + q * 320, 320)])

        plsc.subcore_barrier()

        @pl.loop(0, nb)
        def _(b):
            base = (wid * nb + b) * 512
            pltpu.sync_copy(rowp_hbm.at[pl.ds(base, 512)], row_ref)
            pltpu.sync_copy(colp_hbm.at[pl.ds(base, 512)], col_ref)

            @pl.loop(0, 32)
            def _(k):
                r = row_ref[pl.ds(k * 16, 16)]
                c = col_ref[pl.ds(k * 16, 16)]
                gi = base + k * 16 + _iota16()
                valid = (gi < N_INTER).astype(i32)
                dst = valid * c + (1 - valid) * (N_ITEMS + _iota16())
                kr = k // 8
                kc = (k % 8) * 16
                uidx[kr, pl.ds(kc, 16)] = r * 2 + jj
                didx[kr, pl.ds(kc, 16)] = dst
                cval[kr, pl.ds(kc, 16)] = valid.astype(f32)

            @pl.loop(0, 4)
            def _(q):
                pltpu.sync_copy(ucfh_hbm.at[uidx.at[q]],
                                u_rows.at[pl.ds(q * 128, 128)])

            @pl.loop(0, 4)
            def _(q):
                pltpu.sync_copy(u_rows.at[pl.ds(q * 128, 128)],
                                acc_i.at[didx.at[q]], add=True)

            @pl.when(jj == 0)
            def _():
                @pl.loop(0, 4)
                def _(q):
                    pltpu.sync_copy(cval.at[q], acc_c.at[didx.at[q]], add=True)

        plsc.subcore_barrier()

        @pl.loop(0, 4)
        def _(q):
            off = sid * 1280 + q * 320
            pltpu.sync_copy(acc_i.at[pl.ds(off, 320)], zi)
            pltpu.sync_copy(
                zi, isum_out.at[pl.ds((cid * 2 + jj) * IROWS + off, 320)])

        plsc.subcore_barrier()

    pltpu.sync_copy(acc_c.at[pl.ds(sid * 1280, 1280)], zc)
    pltpu.sync_copy(zc, icnt_out.at[cid].at[pl.ds(sid * 1280, 1280)])


@jax.jit
def _iagg_call(rowp, colp, ucfh):
    k = pl.kernel(
        _iagg_body,
        out_type=(jax.ShapeDtypeStruct((2 * 2 * IROWS, 32), f32),
                  jax.ShapeDtypeStruct((2, IROWS), f32)),
        mesh=_MESH,
        compiler_params=_CP,
        scratch_types=[
            pltpu.VMEM((512,), i32),
            pltpu.VMEM((512,), i32),
            pltpu.VMEM((4, 128), i32),
            pltpu.VMEM((4, 128), i32),
            pltpu.VMEM((4, 128), f32),
            pltpu.VMEM((512, 32), f32),
            pltpu.VMEM((320, 32), f32),
            pltpu.VMEM((1280,), f32),
            pltpu.VMEM_SHARED((IROWS, 32), f32),
            pltpu.VMEM_SHARED((IROWS,), f32),
        ],
    )
    return k(rowp, colp, ucfh)


def kernel(entity_emb, user_emb, user_emb_cf, item_emb_cf, relation_weight,
           W1_w, W1_b, W2_w, W2_b, edge_index, edge_type, interact_mat):
    epad = EPAD - N_EDGES
    pad_h = (jnp.arange(epad, dtype=i32) * 37) % N_ENTITIES
    head = jnp.concatenate([edge_index[0], pad_h])
    tail = jnp.concatenate([edge_index[1], pad_h])
    etype = jnp.pad(edge_type, (0, epad))

    mpad = MPAD - N_INTER
    pad_r = (jnp.arange(mpad, dtype=i32) * 53) % N_USERS
    pad_c = (jnp.arange(mpad, dtype=i32) * 41) % N_ITEMS
    rowp = jnp.concatenate([interact_mat[:, 0], pad_r])
    colp = jnp.concatenate([interact_mat[:, 1], pad_c])

    entq = entity_emb.reshape(N_ENTITIES * 4, 16)
    mq = jnp.repeat(relation_weight, NREP, axis=0).reshape(-1, 16)

    hist2 = _hist_call(head, tail, etype)
    s4 = _scatsum_call(head, tail, etype, entq, mq).reshape(2, 4, SROWS, 16)

    hist = (hist2[0] + hist2[1]).reshape(N_ENTITIES, 2, 16)
    cnt_cross = hist[:, 0, :].sum(1)
    cnt_same = hist[:, 1, :].sum(1)
    sum_same_erel = hist[:, 1, :] @ relation_weight
    rel_sum = (hist[:, 0, :] + hist[:, 1, :]) @ relation_weight
    rel_ = rel_sum / jnp.clip(cnt_cross + cnt_same, 1.0, None)[:, None]

    s_all = s4[:, :, :N_ENTITIES, :].transpose(0, 2, 1, 3).reshape(2, N_ENTITIES, DIM)
    S1 = s_all[0]
    S2 = s_all[1] + sum_same_erel
    agg1 = jax.nn.leaky_relu(S1 / jnp.clip(cnt_cross, 1.0, None)[:, None] @ W1_w.T + W1_b, 0.01) / 2.0
    agg2 = jax.nn.leaky_relu(S2 / jnp.clip(cnt_same, 1.0, None)[:, None] @ W2_w.T + W2_b, 0.01) / 2.0
    entity_agg = agg1 + agg2

    item_emb_kg = entity_emb[:N_ITEMS]
    z = rel_[:N_ITEMS] * item_emb_kg
    v2h = jnp.concatenate([item_emb_kg, item_emb_cf]).reshape(2 * 2 * N_ITEMS, 32)
    z2 = jnp.concatenate([z, item_emb_cf])

    u = user_emb
    ucf = user_emb_cf
    mask_pad = None
    for i in range(MAX_ITER):
        u2 = jnp.concatenate([u, ucf])
        e2, d2 = _dots_call(rowp, colp, u2, z2)
        usum, mask_pad = _uacc_call(rowp, colp, e2, d2.reshape(-1), v2h)
        usum = usum.reshape(2, 2, UROWS, 32)
        u = jnp.concatenate([usum[0, 0, :N_USERS], usum[0, 1, :N_USERS]], axis=1)
        ucf = jnp.concatenate([usum[1, 0, :N_USERS], usum[1, 1, :N_USERS]], axis=1)
        if i < MAX_ITER - 1:
            u = u / jnp.clip(jnp.linalg.norm(u, axis=1, keepdims=True), 1e-12, None)
            ucf = ucf / jnp.clip(jnp.linalg.norm(ucf, axis=1, keepdims=True), 1e-12, None)

    ucfh = user_emb_cf.reshape(2 * N_USERS, 32)
    isum, icnt = _iagg_call(rowp, colp, ucfh)
    isum = isum.reshape(2, 2, IROWS, 32)
    ic = jnp.clip(icnt[0, :N_ITEMS] + icnt[1, :N_ITEMS], 1.0, None)[:, None]
    item_sum = jnp.concatenate([isum[0, 0, :N_ITEMS] + isum[1, 0, :N_ITEMS],
                                isum[0, 1, :N_ITEMS] + isum[1, 1, :N_ITEMS]], axis=1)
    item_agg = item_sum / ic

    return (entity_agg, u, ucf, item_agg, mask_pad[:N_INTER])


# speedup vs baseline: 4.3433x; 1.0387x over previous
import dataclasses
import functools

import jax
import jax.numpy as jnp
from jax import lax
from jax.experimental import pallas as pl
from jax.experimental.pallas import tpu as pltpu
from jax.experimental.pallas import tpu_sc as plsc

N_USERS = 30000
N_ITEMS = 20000
N_ENTITIES = 50000
N_RELATIONS = 16
N_EDGES = 800000
N_INTER = 500000
DIM = 64
GAMMA = 0.6
MAX_ITER = 2

NHIST = N_ENTITIES * 32
EPAD = 16 * 512 * 98
MPAD = 16 * 512 * 62
SROWS = 51200
UROWS = 30720
IROWS = 20480
NREP = 256

f32 = jnp.float32
i32 = jnp.int32

_MESH = plsc.VectorSubcoreMesh(core_axis_name="c", subcore_axis_name="s")

_CP = pltpu.CompilerParams()
if "needs_layout_passes" in pltpu.CompilerParams.__dataclass_fields__:
    _CP = dataclasses.replace(_CP, needs_layout_passes=False)
if "use_tc_tiling_on_sc" in pltpu.CompilerParams.__dataclass_fields__:
    _CP = dataclasses.replace(_CP, use_tc_tiling_on_sc=False)


def _iota16():
    return lax.iota(i32, 16)


def _zero_fill(ref, n):
    @pl.loop(0, n, step=16)
    def _(i):
        ref[pl.ds(i, 16)] = jnp.zeros((16,), f32)


def _zero_fill2(ref, rows, width=16):
    @pl.loop(0, rows)
    def _(r):
        for cc in range(0, width, 16):
            ref[r, pl.ds(cc, 16)] = jnp.zeros((16,), f32)


def _hist_body(head_hbm, tail_hbm, type_hbm, out_hbm,
               h_ref, t_ref, tp_ref, idx_ref, val_ref, zbuf, acc):
    cid = lax.axis_index("c")
    sid = lax.axis_index("s")
    nchunk = NHIST // 6400

    _zero_fill(zbuf, 6400)

    @pl.loop(0, 16)
    def _(it):
        ci = sid + it * 16

        @pl.when(ci < nchunk)
        def _():
            pltpu.sync_copy(zbuf, acc.at[pl.ds(ci * 6400, 6400)])

    plsc.subcore_barrier()

    wid = sid * 2 + cid
    nb = EPAD // (32 * 512)

    @pl.loop(0, nb)
    def _(b):
        base = (wid * nb + b) * 512
        pltpu.sync_copy(head_hbm.at[pl.ds(base, 512)], h_ref)
        pltpu.sync_copy(tail_hbm.at[pl.ds(base, 512)], t_ref)
        pltpu.sync_copy(type_hbm.at[pl.ds(base, 512)], tp_ref)

        @pl.loop(0, 32)
        def _(k):
            h = h_ref[pl.ds(k * 16, 16)]
            t = t_ref[pl.ds(k * 16, 16)]
            tp = tp_ref[pl.ds(k * 16, 16)]
            ha = (h < N_ITEMS).astype(i32)
            ta = (t < N_ITEMS).astype(i32)
            same_i = 1 - (ha ^ ta)
            flat = h * 32 + same_i * 16 + tp
            gi = base + k * 16 + _iota16()
            val = (gi < N_EDGES).astype(f32)
            idx_ref[k // 8, pl.ds((k % 8) * 16, 16)] = flat
            val_ref[k // 8, pl.ds((k % 8) * 16, 16)] = val

        @pl.loop(0, 4)
        def _(j):
            pltpu.sync_copy(val_ref.at[j], acc.at[idx_ref.at[j]], add=True)

    plsc.subcore_barrier()

    @pl.loop(0, 16)
    def _(it):
        ci = sid + it * 16

        @pl.when(ci < nchunk)
        def _():
            pltpu.sync_copy(acc.at[pl.ds(ci * 6400, 6400)], zbuf)
            pltpu.sync_copy(zbuf, out_hbm.at[cid].at[pl.ds(ci * 6400, 6400)])


@jax.jit
def _hist_call(head, tail, etype):
    k = pl.kernel(
        _hist_body,
        out_type=jax.ShapeDtypeStruct((2, NHIST), f32),
        mesh=_MESH,
        compiler_params=_CP,
        scratch_types=[
            pltpu.VMEM((512,), i32),
            pltpu.VMEM((512,), i32),
            pltpu.VMEM((512,), i32),
            pltpu.VMEM((4, 128), i32),
            pltpu.VMEM((4, 128), f32),
            pltpu.VMEM((6400,), f32),
            pltpu.VMEM_SHARED((NHIST,), f32),
        ],
    )
    return k(head, tail, etype)


def _scatsum_body(head_hbm, tail_hbm, type_hbm, entq_hbm, rw_hbm, out_hbm,
                  h_ref, t_ref, tp_ref, gidx, didx,
                  ent_rows, erel_loc, zrow, acc):
    cid = lax.axis_index("c")
    sid = lax.axis_index("s")
    nb = EPAD // (16 * 512)

    pltpu.sync_copy(rw_hbm, erel_loc)

    @pl.loop(0, 4)
    def _(j):
        _zero_fill2(zrow, 1600)

        @pl.loop(0, 2)
        def _(q):
            pltpu.sync_copy(zrow, acc.at[pl.ds(sid * 3200 + q * 1600, 1600)])

        plsc.subcore_barrier()

        @pl.loop(0, nb)
        def _(b):
            base = (sid * nb + b) * 512
            pltpu.sync_copy(head_hbm.at[pl.ds(base, 512)], h_ref)
            pltpu.sync_copy(tail_hbm.at[pl.ds(base, 512)], t_ref)

            @pl.when(cid == 0)
            def _():
                pltpu.sync_copy(type_hbm.at[pl.ds(base, 512)], tp_ref)

            @pl.loop(0, 32)
            def _(k):
                h = h_ref[pl.ds(k * 16, 16)]
                t = t_ref[pl.ds(k * 16, 16)]
                ha = (h < N_ITEMS).astype(i32)
                ta = (t < N_ITEMS).astype(i32)
                same_i = 1 - (ha ^ ta)
                gi = base + k * 16 + _iota16()
                valid = (gi < N_EDGES).astype(i32)
                mine = valid * (1 - (same_i ^ cid))
                kr = k // 8
                kc = (k % 8) * 16
                gidx[kr, pl.ds(kc, 16)] = t * 4 + j
                didx[kr, pl.ds(kc, 16)] = (mine * h
                                           + (1 - mine) * (N_ENTITIES + _iota16()))

            @pl.loop(0, 4)
            def _(q):
                pltpu.sync_copy(entq_hbm.at[gidx.at[q]],
                                ent_rows.at[pl.ds(q * 128, 128)])

            @pl.when(cid == 0)
            def _():
                jofs = j * 16 + _iota16()

                @pl.loop(0, 512)
                def _(r):
                    tpv = plsc.load_gather(tp_ref, [_iota16() * 0 + r])
                    mv = plsc.load_gather(erel_loc, [tpv * 64 + jofs])
                    ent_rows[r, :] = ent_rows[r, :] * mv

            @pl.loop(0, 4)
            def _(q):
                pltpu.sync_copy(ent_rows.at[pl.ds(q * 128, 128)],
                                acc.at[didx.at[q]], add=True)

        plsc.subcore_barrier()

        @pl.loop(0, 2)
        def _(q):
            off = sid * 3200 + q * 1600
            pltpu.sync_copy(acc.at[pl.ds(off, 1600)], zrow)
            pltpu.sync_copy(
                zrow, out_hbm.at[pl.ds((cid * 4 + j) * SROWS + off, 1600)])

        plsc.subcore_barrier()


@jax.jit
def _scatsum_call(head, tail, etype, entq, rw):
    k = pl.kernel(
        _scatsum_body,
        out_type=jax.ShapeDtypeStruct((2 * 4 * SROWS, 16), f32),
        mesh=_MESH,
        compiler_params=_CP,
        scratch_types=[
            pltpu.VMEM((512,), i32),
            pltpu.VMEM((512,), i32),
            pltpu.VMEM((512,), i32),
            pltpu.VMEM((4, 128), i32),
            pltpu.VMEM((4, 128), i32),
            pltpu.VMEM((512, 16), f32),
            pltpu.VMEM((1024,), f32),
            pltpu.VMEM((1600, 16), f32),
            pltpu.VMEM_SHARED((SROWS, 16), f32),
        ],
    )
    return k(head, tail, etype, entq, rw)


def _dots_body(rowp_hbm, colp_hbm, u2_hbm, z2_hbm, e_out, d_out,
               row_ref, col_ref, uidx, zidx, didx,
               u_rows, z_rows, dots, e_lin, zd, acc_d):
    cid = lax.axis_index("c")
    sid = lax.axis_index("s")

    _zero_fill(zd, 1920)
    pltpu.sync_copy(zd, acc_d.at[pl.ds(sid * 1920, 1920)])
    plsc.subcore_barrier()

    nb = MPAD // (16 * 512)

    @pl.loop(0, nb)
    def _(b):
        base = (sid * nb + b) * 512
        pltpu.sync_copy(rowp_hbm.at[pl.ds(base, 512)], row_ref)
        pltpu.sync_copy(colp_hbm.at[pl.ds(base, 512)], col_ref)

        @pl.loop(0, 32)
        def _(k):
            r = row_ref[pl.ds(k * 16, 16)]
            c = col_ref[pl.ds(k * 16, 16)]
            kr = k // 8
            kc = (k % 8) * 16
            uidx[kr, pl.ds(kc, 16)] = cid * N_USERS + r
            zidx[kr, pl.ds(kc, 16)] = cid * N_ITEMS + c
            didx[kr, pl.ds(kc, 16)] = r

        @pl.loop(0, 4)
        def _(q):
            pltpu.sync_copy(u2_hbm.at[uidx.at[q]],
                            u_rows.at[pl.ds(q * 128, 128)])
            pltpu.sync_copy(z2_hbm.at[zidx.at[q]],
                            z_rows.at[pl.ds(q * 128, 128)])

        @pl.loop(0, 32)
        def _(g):
            @pl.loop(0, 16)
            def _(r16):
                rr = g * 16 + r16
                part = (u_rows[rr, pl.ds(0, 16)] * z_rows[rr, pl.ds(0, 16)]
                        + u_rows[rr, pl.ds(16, 16)] * z_rows[rr, pl.ds(16, 16)]
                        + u_rows[rr, pl.ds(32, 16)] * z_rows[rr, pl.ds(32, 16)]
                        + u_rows[rr, pl.ds(48, 16)] * z_rows[rr, pl.ds(48, 16)])
                dots[pl.ds(r16 * 16, 16)] = part

            dv = plsc.load_gather(dots, [_iota16() * 16])
            for cc in range(1, 16):
                dv = dv + plsc.load_gather(dots, [_iota16() * 16 + cc])
            sig = 1.0 / (1.0 + jnp.exp(-dv))
            ev = jnp.exp(sig)
            gi = base + g * 16 + _iota16()
            ev = ev * (gi < N_INTER).astype(f32)
            e_lin[pl.ds(g * 16, 16)] = ev

        pltpu.sync_copy(e_lin, e_out.at[cid].at[pl.ds(base, 512)])

        @pl.loop(0, 4)
        def _(q):
            pltpu.sync_copy(e_lin.at[pl.ds(q * 128, 128)],
                            acc_d.at[didx.at[q]], add=True)

    plsc.subcore_barrier()
    pltpu.sync_copy(acc_d.at[pl.ds(sid * 1920, 1920)], zd)
    pltpu.sync_copy(zd, d_out.at[cid].at[pl.ds(sid * 1920, 1920)])


@jax.jit
def _dots_call(rowp, colp, u2, z2):
    k = pl.kernel(
        _dots_body,
        out_type=(jax.ShapeDtypeStruct((2, MPAD), f32),
                  jax.ShapeDtypeStruct((2, UROWS), f32)),
        mesh=_MESH,
        compiler_params=_CP,
        scratch_types=[
            pltpu.VMEM((512,), i32),
            pltpu.VMEM((512,), i32),
            pltpu.VMEM((4, 128), i32),
            pltpu.VMEM((4, 128), i32),
            pltpu.VMEM((4, 128), i32),
            pltpu.VMEM((512, 64), f32),
            pltpu.VMEM((512, 64), f32),
            pltpu.VMEM((256,), f32),
            pltpu.VMEM((512,), f32),
            pltpu.VMEM((1920,), f32),
            pltpu.VMEM_SHARED((UROWS,), f32),
        ],
    )
    return k(rowp, colp, u2, z2)


def _uacc_body(rowp_hbm, colp_hbm, e2_hbm, dflat_hbm, v2h_hbm,
               usum_out, mask_out,
               row_ref, col_ref, vidx, dpi, dci,
               ep_lin, ec_lin, dp_lin, dc_lin, s_all, mask_lin,
               item_rows, zu, acc_u):
    cid = lax.axis_index("c")
    sid = lax.axis_index("s")

    nb = MPAD // (16 * 512)
    cidf = cid.astype(f32)

    @pl.loop(0, 2)
    def _(jj):
        _zero_fill2(zu, 384, 32)

        @pl.loop(0, 5)
        def _(q):
            pltpu.sync_copy(zu, acc_u.at[pl.ds(sid * 1920 + q * 384, 384)])

        plsc.subcore_barrier()

        @pl.loop(0, nb)
        def _(b):
            base = (sid * nb + b) * 512
            sbase = b * 512
            pltpu.sync_copy(rowp_hbm.at[pl.ds(base, 512)], row_ref)
            pltpu.sync_copy(colp_hbm.at[pl.ds(base, 512)], col_ref)

            @pl.when(jj == 0)
            def _():
                pltpu.sync_copy(e2_hbm.at[0].at[pl.ds(base, 512)], ep_lin)
                pltpu.sync_copy(e2_hbm.at[1].at[pl.ds(base, 512)], ec_lin)

            @pl.loop(0, 32)
            def _(k):
                r = row_ref[pl.ds(k * 16, 16)]
                c = col_ref[pl.ds(k * 16, 16)]
                kr = k // 8
                kc = (k % 8) * 16
                vidx[kr, pl.ds(kc, 16)] = (cid * N_ITEMS + c) * 2 + jj
                dpi[kr, pl.ds(kc, 16)] = r
                dci[kr, pl.ds(kc, 16)] = UROWS + r

            @pl.when(jj == 0)
            def _():
                @pl.loop(0, 4)
                def _(q):
                    pltpu.sync_copy(dflat_hbm.at[dpi.at[q]],
                                    dp_lin.at[pl.ds(q * 128, 128)])
                    pltpu.sync_copy(dflat_hbm.at[dci.at[q]],
                                    dc_lin.at[pl.ds(q * 128, 128)])

            @pl.loop(0, 4)
            def _(q):
                pltpu.sync_copy(v2h_hbm.at[vidx.at[q]],
                                item_rows.at[pl.ds(q * 128, 128)])

            @pl.when(jj == 0)
            def _():
                @pl.loop(0, 32)
                def _(g):
                    sl = pl.ds(g * 16, 16)
                    p = ep_lin[sl] / dp_lin[sl]
                    pcf = ec_lin[sl] / dc_lin[sl]
                    sigp = 1.0 / (1.0 + jnp.exp(-p))
                    sigc = 1.0 / (1.0 + jnp.exp(-pcf))
                    m = (jnp.abs(sigp - sigc) < GAMMA).astype(f32)
                    gi = base + g * 16 + _iota16()
                    validf = (gi < N_INTER).astype(f32)
                    s = (p * (1.0 - cidf) + pcf * cidf) * m * validf
                    s_all[pl.ds(sbase + g * 16, 16)] = s
                    mask_lin[sl] = m.astype(i32)

            @pl.loop(0, 512)
            def _(r):
                sv = plsc.load_gather(s_all, [_iota16() * 0 + (sbase + r)])
                item_rows[r, pl.ds(0, 16)] = item_rows[r, pl.ds(0, 16)] * sv
                item_rows[r, pl.ds(16, 16)] = item_rows[r, pl.ds(16, 16)] * sv

            @pl.loop(0, 4)
            def _(q):
                pltpu.sync_copy(item_rows.at[pl.ds(q * 128, 128)],
                                acc_u.at[dpi.at[q]], add=True)

            @pl.when(cid + jj == 0)
            def _():
                pltpu.sync_copy(mask_lin, mask_out.at[pl.ds(base, 512)])

        plsc.subcore_barrier()

        @pl.loop(0, 5)
        def _(q):
            off = sid * 1920 + q * 384
            pltpu.sync_copy(acc_u.at[pl.ds(off, 384)], zu)
            pltpu.sync_copy(
                zu, usum_out.at[pl.ds((cid * 2 + jj) * UROWS + off, 384)])

        plsc.subcore_barrier()


@jax.jit
def _uacc_call(rowp, colp, e2, dflat, v2h):
    k = pl.kernel(
        _uacc_body,
        out_type=(jax.ShapeDtypeStruct((2 * 2 * UROWS, 32), f32),
                  jax.ShapeDtypeStruct((MPAD,), i32)),
        mesh=_MESH,
        compiler_params=_CP,
        scratch_types=[
            pltpu.VMEM((512,), i32),
            pltpu.VMEM((512,), i32),
            pltpu.VMEM((4, 128), i32),
            pltpu.VMEM((4, 128), i32),
            pltpu.VMEM((4, 128), i32),
            pltpu.VMEM((512,), f32),
            pltpu.VMEM((512,), f32),
            pltpu.VMEM((512,), f32),
            pltpu.VMEM((512,), f32),
            pltpu.VMEM((MPAD // 16,), f32),
            pltpu.VMEM((512,), i32),
            pltpu.VMEM((512, 32), f32),
            pltpu.VMEM((384, 32), f32),
            pltpu.VMEM_SHARED((UROWS, 32), f32),
        ],
    )
    return k(rowp, colp, e2, dflat, v2h)


def _iagg_body(rowp_hbm, colp_hbm, ucfh_hbm, isum_out, icnt_out,
               row_ref, col_ref, uidx, didx, cval,
               u_rows, zi, zc, acc_i, acc_c):
    cid = lax.axis_index("c")
    sid = lax.axis_index("s")

    _zero_fill(zc, 1280)
    pltpu.sync_copy(zc, acc_c.at[pl.ds(sid * 1280, 1280)])

    wid = sid * 2 + cid
    nb = MPAD // (32 * 512)

    @pl.loop(0, 2)
    def _(jj):
        _zero_fill2(zi, 320, 32)

        @pl.loop(0, 4)
        def _(q):
            pltpu.sync_copy(zi, acc_i.at[pl.ds(sid * 1280 + q * 320, 320)])

        plsc.subcore_barrier()

        @pl.loop(0, nb)
        def _(b):
            base = (wid * nb + b) * 512
            pltpu.sync_copy(rowp_hbm.at[pl.ds(base, 512)], row_ref)
            pltpu.sync_copy(colp_hbm.at[pl.ds(base, 512)], col_ref)

            @pl.loop(0, 32)
            def _(k):
                r = row_ref[pl.ds(k * 16, 16)]
                c = col_ref[pl.ds(k * 16, 16)]
                gi = base + k * 16 + _iota16()
                valid = (gi < N_INTER).astype(i32)
                dst = valid * c + (1 - valid) * (N_ITEMS + _iota16())
                kr = k // 8
                kc = (k % 8) * 16
                uidx[kr, pl.ds(kc, 16)] = r * 2 + jj
                didx[kr, pl.ds(kc, 16)] = dst
                cval[kr, pl.ds(kc, 16)] = valid.astype(f32)

            @pl.loop(0, 4)
            def _(q):
                pltpu.sync_copy(ucfh_hbm.at[uidx.at[q]],
                                u_rows.at[pl.ds(q * 128, 128)])

            @pl.loop(0, 4)
            def _(q):
                pltpu.sync_copy(u_rows.at[pl.ds(q * 128, 128)],
                                acc_i.at[didx.at[q]], add=True)

            @pl.when(jj == 0)
            def _():
                @pl.loop(0, 4)
                def _(q):
                    pltpu.sync_copy(cval.at[q], acc_c.at[didx.at[q]], add=True)

        plsc.subcore_barrier()

        @pl.loop(0, 4)
        def _(q):
            off = sid * 1280 + q * 320
            pltpu.sync_copy(acc_i.at[pl.ds(off, 320)], zi)
            pltpu.sync_copy(
                zi, isum_out.at[pl.ds((cid * 2 + jj) * IROWS + off, 320)])

        plsc.subcore_barrier()

    pltpu.sync_copy(acc_c.at[pl.ds(sid * 1280, 1280)], zc)
    pltpu.sync_copy(zc, icnt_out.at[cid].at[pl.ds(sid * 1280, 1280)])


@jax.jit
def _iagg_call(rowp, colp, ucfh):
    k = pl.kernel(
        _iagg_body,
        out_type=(jax.ShapeDtypeStruct((2 * 2 * IROWS, 32), f32),
                  jax.ShapeDtypeStruct((2, IROWS), f32)),
        mesh=_MESH,
        compiler_params=_CP,
        scratch_types=[
            pltpu.VMEM((512,), i32),
            pltpu.VMEM((512,), i32),
            pltpu.VMEM((4, 128), i32),
            pltpu.VMEM((4, 128), i32),
            pltpu.VMEM((4, 128), f32),
            pltpu.VMEM((512, 32), f32),
            pltpu.VMEM((320, 32), f32),
            pltpu.VMEM((1280,), f32),
            pltpu.VMEM_SHARED((IROWS, 32), f32),
            pltpu.VMEM_SHARED((IROWS,), f32),
        ],
    )
    return k(rowp, colp, ucfh)


def kernel(entity_emb, user_emb, user_emb_cf, item_emb_cf, relation_weight,
           W1_w, W1_b, W2_w, W2_b, edge_index, edge_type, interact_mat):
    epad = EPAD - N_EDGES
    pad_h = (jnp.arange(epad, dtype=i32) * 37) % N_ENTITIES
    head = jnp.concatenate([edge_index[0], pad_h])
    tail = jnp.concatenate([edge_index[1], pad_h])
    etype = jnp.pad(edge_type, (0, epad))

    mpad = MPAD - N_INTER
    pad_r = (jnp.arange(mpad, dtype=i32) * 53) % N_USERS
    pad_c = (jnp.arange(mpad, dtype=i32) * 41) % N_ITEMS
    rowp = jnp.concatenate([interact_mat[:, 0], pad_r])
    colp = jnp.concatenate([interact_mat[:, 1], pad_c])

    entq = entity_emb.reshape(N_ENTITIES * 4, 16)

    hist2 = _hist_call(head, tail, etype)
    s4 = _scatsum_call(head, tail, etype, entq,
                       relation_weight.reshape(-1)).reshape(2, 4, SROWS, 16)

    hist = (hist2[0] + hist2[1]).reshape(N_ENTITIES, 2, 16)
    cnt_cross = hist[:, 0, :].sum(1)
    cnt_same = hist[:, 1, :].sum(1)
    sum_same_erel = hist[:, 1, :] @ relation_weight
    rel_sum = (hist[:, 0, :] + hist[:, 1, :]) @ relation_weight
    rel_ = rel_sum / jnp.clip(cnt_cross + cnt_same, 1.0, None)[:, None]

    s_all = s4[:, :, :N_ENTITIES, :].transpose(0, 2, 1, 3).reshape(2, N_ENTITIES, DIM)
    S1 = s_all[0]
    S2 = s_all[1] + sum_same_erel
    agg1 = jax.nn.leaky_relu(S1 / jnp.clip(cnt_cross, 1.0, None)[:, None] @ W1_w.T + W1_b, 0.01) / 2.0
    agg2 = jax.nn.leaky_relu(S2 / jnp.clip(cnt_same, 1.0, None)[:, None] @ W2_w.T + W2_b, 0.01) / 2.0
    entity_agg = agg1 + agg2

    item_emb_kg = entity_emb[:N_ITEMS]
    z = rel_[:N_ITEMS] * item_emb_kg
    v2h = jnp.concatenate([item_emb_kg, item_emb_cf]).reshape(2 * 2 * N_ITEMS, 32)
    z2 = jnp.concatenate([z, item_emb_cf])

    u = user_emb
    ucf = user_emb_cf
    mask_pad = None
    for i in range(MAX_ITER):
        u2 = jnp.concatenate([u, ucf])
        e2, d2 = _dots_call(rowp, colp, u2, z2)
        usum, mask_pad = _uacc_call(rowp, colp, e2, d2.reshape(-1), v2h)
        usum = usum.reshape(2, 2, UROWS, 32)
        u = jnp.concatenate([usum[0, 0, :N_USERS], usum[0, 1, :N_USERS]], axis=1)
        ucf = jnp.concatenate([usum[1, 0, :N_USERS], usum[1, 1, :N_USERS]], axis=1)
        if i < MAX_ITER - 1:
            u = u / jnp.clip(jnp.linalg.norm(u, axis=1, keepdims=True), 1e-12, None)
            ucf = ucf / jnp.clip(jnp.linalg.norm(ucf, axis=1, keepdims=True), 1e-12, None)

    ucfh = user_emb_cf.reshape(2 * N_USERS, 32)
    isum, icnt = _iagg_call(rowp, colp, ucfh)
    isum = isum.reshape(2, 2, IROWS, 32)
    ic = jnp.clip(icnt[0, :N_ITEMS] + icnt[1, :N_ITEMS], 1.0, None)[:, None]
    item_sum = jnp.concatenate([isum[0, 0, :N_ITEMS] + isum[1, 0, :N_ITEMS],
                                isum[0, 1, :N_ITEMS] + isum[1, 1, :N_ITEMS]], axis=1)
    item_agg = item_sum / ic

    return (entity_agg, u, ucf, item_agg, mask_pad[:N_INTER])


# speedup vs baseline: 6.2006x; 1.4276x over previous
import dataclasses
import functools

import jax
import jax.numpy as jnp
from jax import lax
from jax.experimental import pallas as pl
from jax.experimental.pallas import tpu as pltpu
from jax.experimental.pallas import tpu_sc as plsc

N_USERS = 30000
N_ITEMS = 20000
N_ENTITIES = 50000
N_RELATIONS = 16
N_EDGES = 800000
N_INTER = 500000
DIM = 64
GAMMA = 0.6
MAX_ITER = 2

NHIST = N_ENTITIES * 32
EPAD = 16 * 512 * 98
MPAD = 16 * 512 * 62
SROWS = 51200
UROWS = 30720
IROWS = 20480
NREP = 256

f32 = jnp.float32
i32 = jnp.int32

_MESH = plsc.VectorSubcoreMesh(core_axis_name="c", subcore_axis_name="s")

_CP = pltpu.CompilerParams()
if "needs_layout_passes" in pltpu.CompilerParams.__dataclass_fields__:
    _CP = dataclasses.replace(_CP, needs_layout_passes=False)
if "use_tc_tiling_on_sc" in pltpu.CompilerParams.__dataclass_fields__:
    _CP = dataclasses.replace(_CP, use_tc_tiling_on_sc=False)


def _iota16():
    return lax.iota(i32, 16)


def _zero_fill(ref, n):
    @pl.loop(0, n, step=16)
    def _(i):
        ref[pl.ds(i, 16)] = jnp.zeros((16,), f32)


def _zero_fill2(ref, rows, width=16):
    @pl.loop(0, rows)
    def _(r):
        for cc in range(0, width, 16):
            ref[r, pl.ds(cc, 16)] = jnp.zeros((16,), f32)


def _hist_body(head_hbm, tail_hbm, type_hbm, out_hbm,
               h_ref, t_ref, tp_ref, idx_ref, val_ref, zbuf, acc):
    cid = lax.axis_index("c")
    sid = lax.axis_index("s")
    nchunk = NHIST // 6400

    _zero_fill(zbuf, 6400)

    @pl.loop(0, 16)
    def _(it):
        ci = sid + it * 16

        @pl.when(ci < nchunk)
        def _():
            pltpu.sync_copy(zbuf, acc.at[pl.ds(ci * 6400, 6400)])

    plsc.subcore_barrier()

    wid = sid * 2 + cid
    nb = EPAD // (32 * 512)

    @pl.loop(0, nb)
    def _(b):
        base = (wid * nb + b) * 512
        pltpu.sync_copy(head_hbm.at[pl.ds(base, 512)], h_ref)
        pltpu.sync_copy(tail_hbm.at[pl.ds(base, 512)], t_ref)
        pltpu.sync_copy(type_hbm.at[pl.ds(base, 512)], tp_ref)

        @pl.loop(0, 32)
        def _(k):
            h = h_ref[pl.ds(k * 16, 16)]
            t = t_ref[pl.ds(k * 16, 16)]
            tp = tp_ref[pl.ds(k * 16, 16)]
            ha = (h < N_ITEMS).astype(i32)
            ta = (t < N_ITEMS).astype(i32)
            same_i = 1 - (ha ^ ta)
            flat = h * 32 + same_i * 16 + tp
            gi = base + k * 16 + _iota16()
            val = (gi < N_EDGES).astype(f32)
            idx_ref[k // 8, pl.ds((k % 8) * 16, 16)] = flat
            val_ref[k // 8, pl.ds((k % 8) * 16, 16)] = val

        @pl.loop(0, 4)
        def _(j):
            pltpu.sync_copy(val_ref.at[j], acc.at[idx_ref.at[j]], add=True)

    plsc.subcore_barrier()

    @pl.loop(0, 16)
    def _(it):
        ci = sid + it * 16

        @pl.when(ci < nchunk)
        def _():
            pltpu.sync_copy(acc.at[pl.ds(ci * 6400, 6400)], zbuf)
            pltpu.sync_copy(zbuf, out_hbm.at[cid].at[pl.ds(ci * 6400, 6400)])


@jax.jit
def _hist_call(head, tail, etype):
    k = pl.kernel(
        _hist_body,
        out_type=jax.ShapeDtypeStruct((2, NHIST), f32),
        mesh=_MESH,
        compiler_params=_CP,
        scratch_types=[
            pltpu.VMEM((512,), i32),
            pltpu.VMEM((512,), i32),
            pltpu.VMEM((512,), i32),
            pltpu.VMEM((4, 128), i32),
            pltpu.VMEM((4, 128), f32),
            pltpu.VMEM((6400,), f32),
            pltpu.VMEM_SHARED((NHIST,), f32),
        ],
    )
    return k(head, tail, etype)


def _scatsum_body(head_hbm, tail_hbm, type_hbm, tab_hbm, out_hbm,
                  h_ref, t_ref, tp_ref, gidx, didx,
                  ent_rows, zrow, acc, gsem, ssem, esem):
    cid = lax.axis_index("c")
    sid = lax.axis_index("s")
    nb = EPAD // (16 * 512)

    @pl.loop(0, 4)
    def _(j):
        _zero_fill2(zrow, 1600)

        @pl.loop(0, 2)
        def _(q):
            pltpu.sync_copy(zrow, acc.at[pl.ds(sid * 3200 + q * 1600, 1600)])

        plsc.subcore_barrier()

        @pl.loop(0, nb)
        def _(b):
            base = (sid * nb + b) * 512
            e1 = pltpu.async_copy(head_hbm.at[pl.ds(base, 512)], h_ref, esem)
            e2 = pltpu.async_copy(tail_hbm.at[pl.ds(base, 512)], t_ref, esem)
            e3 = pltpu.async_copy(type_hbm.at[pl.ds(base, 512)], tp_ref, esem)
            e1.wait()
            e2.wait()
            e3.wait()

            @pl.loop(0, 32)
            def _(k):
                h = h_ref[pl.ds(k * 16, 16)]
                t = t_ref[pl.ds(k * 16, 16)]
                tp = tp_ref[pl.ds(k * 16, 16)]
                ha = (h < N_ITEMS).astype(i32)
                ta = (t < N_ITEMS).astype(i32)
                same_i = 1 - (ha ^ ta)
                gi = base + k * 16 + _iota16()
                valid = (gi < N_EDGES).astype(i32)
                mine = valid * (1 - (same_i ^ cid))
                kr = k // 8
                kc = (k % 8) * 16
                gidx[kr, pl.ds(kc, 16)] = (t + (1 - cid) * (N_ENTITIES
                                                            + tp * N_ENTITIES)) * 4 + j
                didx[kr, pl.ds(kc, 16)] = (mine * h
                                           + (1 - mine) * (N_ENTITIES + _iota16()))

            gds = []
            for q in range(4):
                gds.append(pltpu.async_copy(tab_hbm.at[gidx.at[q]],
                                            ent_rows.at[pl.ds(q * 128, 128)],
                                            gsem))
            for gd in gds:
                gd.wait()

            sds = []
            for q in range(4):
                sds.append(pltpu.async_copy(ent_rows.at[pl.ds(q * 128, 128)],
                                            acc.at[didx.at[q]], ssem, add=True))
            for sd in sds:
                sd.wait()

        plsc.subcore_barrier()

        @pl.loop(0, 2)
        def _(q):
            off = sid * 3200 + q * 1600
            pltpu.sync_copy(acc.at[pl.ds(off, 1600)], zrow)
            pltpu.sync_copy(
                zrow, out_hbm.at[pl.ds((cid * 4 + j) * SROWS + off, 1600)])

        plsc.subcore_barrier()


@jax.jit
def _scatsum_call(head, tail, etype, tab):
    k = pl.kernel(
        _scatsum_body,
        out_type=jax.ShapeDtypeStruct((2 * 4 * SROWS, 16), f32),
        mesh=_MESH,
        compiler_params=_CP,
        scratch_types=[
            pltpu.VMEM((512,), i32),
            pltpu.VMEM((512,), i32),
            pltpu.VMEM((512,), i32),
            pltpu.VMEM((4, 128), i32),
            pltpu.VMEM((4, 128), i32),
            pltpu.VMEM((512, 16), f32),
            pltpu.VMEM((1600, 16), f32),
            pltpu.VMEM_SHARED((SROWS, 16), f32),
            pltpu.SemaphoreType.DMA,
            pltpu.SemaphoreType.DMA,
            pltpu.SemaphoreType.DMA,
        ],
    )
    return k(head, tail, etype, tab)


def _dots_body(rowp_hbm, colp_hbm, u2_hbm, z2_hbm, e_out, d_out,
               row_ref, col_ref, uidx, zidx, didx,
               u_rows, z_rows, dots, e_lin, zd, acc_d):
    cid = lax.axis_index("c")
    sid = lax.axis_index("s")

    _zero_fill(zd, 1920)
    pltpu.sync_copy(zd, acc_d.at[pl.ds(sid * 1920, 1920)])
    plsc.subcore_barrier()

    nb = MPAD // (16 * 512)

    @pl.loop(0, nb)
    def _(b):
        base = (sid * nb + b) * 512
        pltpu.sync_copy(rowp_hbm.at[pl.ds(base, 512)], row_ref)
        pltpu.sync_copy(colp_hbm.at[pl.ds(base, 512)], col_ref)

        @pl.loop(0, 32)
        def _(k):
            r = row_ref[pl.ds(k * 16, 16)]
            c = col_ref[pl.ds(k * 16, 16)]
            kr = k // 8
            kc = (k % 8) * 16
            uidx[kr, pl.ds(kc, 16)] = cid * N_USERS + r
            zidx[kr, pl.ds(kc, 16)] = cid * N_ITEMS + c
            didx[kr, pl.ds(kc, 16)] = r

        @pl.loop(0, 4)
        def _(q):
            pltpu.sync_copy(u2_hbm.at[uidx.at[q]],
                            u_rows.at[pl.ds(q * 128, 128)])
            pltpu.sync_copy(z2_hbm.at[zidx.at[q]],
                            z_rows.at[pl.ds(q * 128, 128)])

        @pl.loop(0, 32)
        def _(g):
            @pl.loop(0, 16)
            def _(r16):
                rr = g * 16 + r16
                part = (u_rows[rr, pl.ds(0, 16)] * z_rows[rr, pl.ds(0, 16)]
                        + u_rows[rr, pl.ds(16, 16)] * z_rows[rr, pl.ds(16, 16)]
                        + u_rows[rr, pl.ds(32, 16)] * z_rows[rr, pl.ds(32, 16)]
                        + u_rows[rr, pl.ds(48, 16)] * z_rows[rr, pl.ds(48, 16)])
                dots[pl.ds(r16 * 16, 16)] = part

            dv = plsc.load_gather(dots, [_iota16() * 16])
            for cc in range(1, 16):
                dv = dv + plsc.load_gather(dots, [_iota16() * 16 + cc])
            sig = 1.0 / (1.0 + jnp.exp(-dv))
            ev = jnp.exp(sig)
            gi = base + g * 16 + _iota16()
            ev = ev * (gi < N_INTER).astype(f32)
            e_lin[pl.ds(g * 16, 16)] = ev

        pltpu.sync_copy(e_lin, e_out.at[cid].at[pl.ds(base, 512)])

        @pl.loop(0, 4)
        def _(q):
            pltpu.sync_copy(e_lin.at[pl.ds(q * 128, 128)],
                            acc_d.at[didx.at[q]], add=True)

    plsc.subcore_barrier()
    pltpu.sync_copy(acc_d.at[pl.ds(sid * 1920, 1920)], zd)
    pltpu.sync_copy(zd, d_out.at[cid].at[pl.ds(sid * 1920, 1920)])


@jax.jit
def _dots_call(rowp, colp, u2, z2):
    k = pl.kernel(
        _dots_body,
        out_type=(jax.ShapeDtypeStruct((2, MPAD), f32),
                  jax.ShapeDtypeStruct((2, UROWS), f32)),
        mesh=_MESH,
        compiler_params=_CP,
        scratch_types=[
            pltpu.VMEM((512,), i32),
            pltpu.VMEM((512,), i32),
            pltpu.VMEM((4, 128), i32),
            pltpu.VMEM((4, 128), i32),
            pltpu.VMEM((4, 128), i32),
            pltpu.VMEM((512, 64), f32),
            pltpu.VMEM((512, 64), f32),
            pltpu.VMEM((256,), f32),
            pltpu.VMEM((512,), f32),
            pltpu.VMEM((1920,), f32),
            pltpu.VMEM_SHARED((UROWS,), f32),
        ],
    )
    return k(rowp, colp, u2, z2)


def _uacc_body(rowp_hbm, colp_hbm, e2_hbm, dflat_hbm, v2h_hbm,
               usum_out, mask_out,
               row_ref, col_ref, vidx, dpi, dci,
               ep_lin, ec_lin, dp_lin, dc_lin, s_all, mask_lin,
               item_rows, zu, acc_u):
    cid = lax.axis_index("c")
    sid = lax.axis_index("s")

    nb = MPAD // (16 * 512)
    cidf = cid.astype(f32)

    @pl.loop(0, 2)
    def _(jj):
        _zero_fill2(zu, 384, 32)

        @pl.loop(0, 5)
        def _(q):
            pltpu.sync_copy(zu, acc_u.at[pl.ds(sid * 1920 + q * 384, 384)])

        plsc.subcore_barrier()

        @pl.loop(0, nb)
        def _(b):
            base = (sid * nb + b) * 512
            sbase = b * 512
            pltpu.sync_copy(rowp_hbm.at[pl.ds(base, 512)], row_ref)
            pltpu.sync_copy(colp_hbm.at[pl.ds(base, 512)], col_ref)

            @pl.when(jj == 0)
            def _():
                pltpu.sync_copy(e2_hbm.at[0].at[pl.ds(base, 512)], ep_lin)
                pltpu.sync_copy(e2_hbm.at[1].at[pl.ds(base, 512)], ec_lin)

            @pl.loop(0, 32)
            def _(k):
                r = row_ref[pl.ds(k * 16, 16)]
                c = col_ref[pl.ds(k * 16, 16)]
                kr = k // 8
                kc = (k % 8) * 16
                vidx[kr, pl.ds(kc, 16)] = (cid * N_ITEMS + c) * 2 + jj
                dpi[kr, pl.ds(kc, 16)] = r
                dci[kr, pl.ds(kc, 16)] = UROWS + r

            @pl.when(jj == 0)
            def _():
                @pl.loop(0, 4)
                def _(q):
                    pltpu.sync_copy(dflat_hbm.at[dpi.at[q]],
                                    dp_lin.at[pl.ds(q * 128, 128)])
                    pltpu.sync_copy(dflat_hbm.at[dci.at[q]],
                                    dc_lin.at[pl.ds(q * 128, 128)])

            @pl.loop(0, 4)
            def _(q):
                pltpu.sync_copy(v2h_hbm.at[vidx.at[q]],
                                item_rows.at[pl.ds(q * 128, 128)])

            @pl.when(jj == 0)
            def _():
                @pl.loop(0, 32)
                def _(g):
                    sl = pl.ds(g * 16, 16)
                    p = ep_lin[sl] / dp_lin[sl]
                    pcf = ec_lin[sl] / dc_lin[sl]
                    sigp = 1.0 / (1.0 + jnp.exp(-p))
                    sigc = 1.0 / (1.0 + jnp.exp(-pcf))
                    m = (jnp.abs(sigp - sigc) < GAMMA).astype(f32)
                    gi = base + g * 16 + _iota16()
                    validf = (gi < N_INTER).astype(f32)
                    s = (p * (1.0 - cidf) + pcf * cidf) * m * validf
                    s_all[pl.ds(sbase + g * 16, 16)] = s
                    mask_lin[sl] = m.astype(i32)

            @pl.loop(0, 512)
            def _(r):
                sv = plsc.load_gather(s_all, [_iota16() * 0 + (sbase + r)])
                item_rows[r, pl.ds(0, 16)] = item_rows[r, pl.ds(0, 16)] * sv
                item_rows[r, pl.ds(16, 16)] = item_rows[r, pl.ds(16, 16)] * sv

            @pl.loop(0, 4)
            def _(q):
                pltpu.sync_copy(item_rows.at[pl.ds(q * 128, 128)],
                                acc_u.at[dpi.at[q]], add=True)

            @pl.when(cid + jj == 0)
            def _():
                pltpu.sync_copy(mask_lin, mask_out.at[pl.ds(base, 512)])

        plsc.subcore_barrier()

        @pl.loop(0, 5)
        def _(q):
            off = sid * 1920 + q * 384
            pltpu.sync_copy(acc_u.at[pl.ds(off, 384)], zu)
            pltpu.sync_copy(
                zu, usum_out.at[pl.ds((cid * 2 + jj) * UROWS + off, 384)])

        plsc.subcore_barrier()


@jax.jit
def _uacc_call(rowp, colp, e2, dflat, v2h):
    k = pl.kernel(
        _uacc_body,
        out_type=(jax.ShapeDtypeStruct((2 * 2 * UROWS, 32), f32),
                  jax.ShapeDtypeStruct((MPAD,), i32)),
        mesh=_MESH,
        compiler_params=_CP,
        scratch_types=[
            pltpu.VMEM((512,), i32),
            pltpu.VMEM((512,), i32),
            pltpu.VMEM((4, 128), i32),
            pltpu.VMEM((4, 128), i32),
            pltpu.VMEM((4, 128), i32),
            pltpu.VMEM((512,), f32),
            pltpu.VMEM((512,), f32),
            pltpu.VMEM((512,), f32),
            pltpu.VMEM((512,), f32),
            pltpu.VMEM((MPAD // 16,), f32),
            pltpu.VMEM((512,), i32),
            pltpu.VMEM((512, 32), f32),
            pltpu.VMEM((384, 32), f32),
            pltpu.VMEM_SHARED((UROWS, 32), f32),
        ],
    )
    return k(rowp, colp, e2, dflat, v2h)


def _iagg_body(rowp_hbm, colp_hbm, ucfh_hbm, isum_out, icnt_out,
               row_ref, col_ref, uidx, didx, cval,
               u_rows, zi, zc, acc_i, acc_c):
    cid = lax.axis_index("c")
    sid = lax.axis_index("s")

    _zero_fill(zc, 1280)
    pltpu.sync_copy(zc, acc_c.at[pl.ds(sid * 1280, 1280)])

    wid = sid * 2 + cid
    nb = MPAD // (32 * 512)

    @pl.loop(0, 2)
    def _(jj):
        _zero_fill2(zi, 320, 32)

        @pl.loop(0, 4)
        def _(q):
            pltpu.sync_copy(zi, acc_i.at[pl.ds(sid * 1280 + q * 320, 320)])

        plsc.subcore_barrier()

        @pl.loop(0, nb)
        def _(b):
            base = (wid * nb + b) * 512
            pltpu.sync_copy(rowp_hbm.at[pl.ds(base, 512)], row_ref)
            pltpu.sync_copy(colp_hbm.at[pl.ds(base, 512)], col_ref)

            @pl.loop(0, 32)
            def _(k):
                r = row_ref[pl.ds(k * 16, 16)]
                c = col_ref[pl.ds(k * 16, 16)]
                gi = base + k * 16 + _iota16()
                valid = (gi < N_INTER).astype(i32)
                dst = valid * c + (1 - valid) * (N_ITEMS + _iota16())
                kr = k // 8
                kc = (k % 8) * 16
                uidx[kr, pl.ds(kc, 16)] = r * 2 + jj
                didx[kr, pl.ds(kc, 16)] = dst
                cval[kr, pl.ds(kc, 16)] = valid.astype(f32)

            @pl.loop(0, 4)
            def _(q):
                pltpu.sync_copy(ucfh_hbm.at[uidx.at[q]],
                                u_rows.at[pl.ds(q * 128, 128)])

            @pl.loop(0, 4)
            def _(q):
                pltpu.sync_copy(u_rows.at[pl.ds(q * 128, 128)],
                                acc_i.at[didx.at[q]], add=True)

            @pl.when(jj == 0)
            def _():
                @pl.loop(0, 4)
                def _(q):
                    pltpu.sync_copy(cval.at[q], acc_c.at[didx.at[q]], add=True)

        plsc.subcore_barrier()

        @pl.loop(0, 4)
        def _(q):
            off = sid * 1280 + q * 320
            pltpu.sync_copy(acc_i.at[pl.ds(off, 320)], zi)
            pltpu.sync_copy(
                zi, isum_out.at[pl.ds((cid * 2 + jj) * IROWS + off, 320)])

        plsc.subcore_barrier()

    pltpu.sync_copy(acc_c.at[pl.ds(sid * 1280, 1280)], zc)
    pltpu.sync_copy(zc, icnt_out.at[cid].at[pl.ds(sid * 1280, 1280)])


@jax.jit
def _iagg_call(rowp, colp, ucfh):
    k = pl.kernel(
        _iagg_body,
        out_type=(jax.ShapeDtypeStruct((2 * 2 * IROWS, 32), f32),
                  jax.ShapeDtypeStruct((2, IROWS), f32)),
        mesh=_MESH,
        compiler_params=_CP,
        scratch_types=[
            pltpu.VMEM((512,), i32),
            pltpu.VMEM((512,), i32),
            pltpu.VMEM((4, 128), i32),
            pltpu.VMEM((4, 128), i32),
            pltpu.VMEM((4, 128), f32),
            pltpu.VMEM((512, 32), f32),
            pltpu.VMEM((320, 32), f32),
            pltpu.VMEM((1280,), f32),
            pltpu.VMEM_SHARED((IROWS, 32), f32),
            pltpu.VMEM_SHARED((IROWS,), f32),
        ],
    )
    return k(rowp, colp, ucfh)


def kernel(entity_emb, user_emb, user_emb_cf, item_emb_cf, relation_weight,
           W1_w, W1_b, W2_w, W2_b, edge_index, edge_type, interact_mat):
    epad = EPAD - N_EDGES
    pad_h = (jnp.arange(epad, dtype=i32) * 37) % N_ENTITIES
    head = jnp.concatenate([edge_index[0], pad_h])
    tail = jnp.concatenate([edge_index[1], pad_h])
    etype = jnp.pad(edge_type, (0, epad))

    mpad = MPAD - N_INTER
    pad_r = (jnp.arange(mpad, dtype=i32) * 53) % N_USERS
    pad_c = (jnp.arange(mpad, dtype=i32) * 41) % N_ITEMS
    rowp = jnp.concatenate([interact_mat[:, 0], pad_r])
    colp = jnp.concatenate([interact_mat[:, 1], pad_c])

    tab = jnp.concatenate(
        [entity_emb[None], relation_weight[:, None, :] * entity_emb[None]],
        axis=0).reshape(-1, 16)

    hist2 = _hist_call(head, tail, etype)
    s4 = _scatsum_call(head, tail, etype, tab).reshape(2, 4, SROWS, 16)

    hist = (hist2[0] + hist2[1]).reshape(N_ENTITIES, 2, 16)
    cnt_cross = hist[:, 0, :].sum(1)
    cnt_same = hist[:, 1, :].sum(1)
    sum_same_erel = hist[:, 1, :] @ relation_weight
    rel_sum = (hist[:, 0, :] + hist[:, 1, :]) @ relation_weight
    rel_ = rel_sum / jnp.clip(cnt_cross + cnt_same, 1.0, None)[:, None]

    s_all = s4[:, :, :N_ENTITIES, :].transpose(0, 2, 1, 3).reshape(2, N_ENTITIES, DIM)
    S1 = s_all[0]
    S2 = s_all[1] + sum_same_erel
    agg1 = jax.nn.leaky_relu(S1 / jnp.clip(cnt_cross, 1.0, None)[:, None] @ W1_w.T + W1_b, 0.01) / 2.0
    agg2 = jax.nn.leaky_relu(S2 / jnp.clip(cnt_same, 1.0, None)[:, None] @ W2_w.T + W2_b, 0.01) / 2.0
    entity_agg = agg1 + agg2

    item_emb_kg = entity_emb[:N_ITEMS]
    z = rel_[:N_ITEMS] * item_emb_kg
    v2h = jnp.concatenate([item_emb_kg, item_emb_cf]).reshape(2 * 2 * N_ITEMS, 32)
    z2 = jnp.concatenate([z, item_emb_cf])

    u = user_emb
    ucf = user_emb_cf
    mask_pad = None
    for i in range(MAX_ITER):
        u2 = jnp.concatenate([u, ucf])
        e2, d2 = _dots_call(rowp, colp, u2, z2)
        usum, mask_pad = _uacc_call(rowp, colp, e2, d2.reshape(-1), v2h)
        usum = usum.reshape(2, 2, UROWS, 32)
        u = jnp.concatenate([usum[0, 0, :N_USERS], usum[0, 1, :N_USERS]], axis=1)
        ucf = jnp.concatenate([usum[1, 0, :N_USERS], usum[1, 1, :N_USERS]], axis=1)
        if i < MAX_ITER - 1:
            u = u / jnp.clip(jnp.linalg.norm(u, axis=1, keepdims=True), 1e-12, None)
            ucf = ucf / jnp.clip(jnp.linalg.norm(ucf, axis=1, keepdims=True), 1e-12, None)

    ucfh = user_emb_cf.reshape(2 * N_USERS, 32)
    isum, icnt = _iagg_call(rowp, colp, ucfh)
    isum = isum.reshape(2, 2, IROWS, 32)
    ic = jnp.clip(icnt[0, :N_ITEMS] + icnt[1, :N_ITEMS], 1.0, None)[:, None]
    item_sum = jnp.concatenate([isum[0, 0, :N_ITEMS] + isum[1, 0, :N_ITEMS],
                                isum[0, 1, :N_ITEMS] + isum[1, 1, :N_ITEMS]], axis=1)
    item_agg = item_sum / ic

    return (entity_agg, u, ucf, item_agg, mask_pad[:N_INTER])


# speedup vs baseline: 8.2811x; 1.3355x over previous
import dataclasses
import functools

import jax
import jax.numpy as jnp
from jax import lax
from jax.experimental import pallas as pl
from jax.experimental.pallas import tpu as pltpu
from jax.experimental.pallas import tpu_sc as plsc

N_USERS = 30000
N_ITEMS = 20000
N_ENTITIES = 50000
N_RELATIONS = 16
N_EDGES = 800000
N_INTER = 500000
DIM = 64
GAMMA = 0.6
MAX_ITER = 2

NHIST = N_ENTITIES * 32
EPAD = 16 * 512 * 98
MPAD = 16 * 512 * 62
SROWS = 51200
UROWS = 30720
IROWS = 20480
NREP = 256

f32 = jnp.float32
i32 = jnp.int32

_MESH = plsc.VectorSubcoreMesh(core_axis_name="c", subcore_axis_name="s")

_CP = pltpu.CompilerParams()
if "needs_layout_passes" in pltpu.CompilerParams.__dataclass_fields__:
    _CP = dataclasses.replace(_CP, needs_layout_passes=False)
if "use_tc_tiling_on_sc" in pltpu.CompilerParams.__dataclass_fields__:
    _CP = dataclasses.replace(_CP, use_tc_tiling_on_sc=False)


def _iota16():
    return lax.iota(i32, 16)


def _zero_fill(ref, n):
    @pl.loop(0, n, step=16)
    def _(i):
        ref[pl.ds(i, 16)] = jnp.zeros((16,), f32)


def _zero_fill2(ref, rows, width=16):
    @pl.loop(0, rows)
    def _(r):
        for cc in range(0, width, 16):
            ref[r, pl.ds(cc, 16)] = jnp.zeros((16,), f32)


def _hist_body(head_hbm, tail_hbm, type_hbm, out_hbm,
               h_ref, t_ref, tp_ref, idx_ref, val_ref, zbuf, acc, esem, ssem):
    cid = lax.axis_index("c")
    sid = lax.axis_index("s")
    nchunk = NHIST // 6400

    _zero_fill(zbuf, 6400)

    @pl.loop(0, 16)
    def _(it):
        ci = sid + it * 16

        @pl.when(ci < nchunk)
        def _():
            pltpu.sync_copy(zbuf, acc.at[pl.ds(ci * 6400, 6400)])

    plsc.subcore_barrier()

    wid = sid * 2 + cid
    nb = EPAD // (32 * 512)

    @pl.loop(0, nb)
    def _(b):
        base = (wid * nb + b) * 512
        eds = [pltpu.async_copy(head_hbm.at[pl.ds(base, 512)], h_ref, esem),
               pltpu.async_copy(tail_hbm.at[pl.ds(base, 512)], t_ref, esem),
               pltpu.async_copy(type_hbm.at[pl.ds(base, 512)], tp_ref, esem)]
        for ed in eds:
            ed.wait()

        @pl.loop(0, 32)
        def _(k):
            h = h_ref[pl.ds(k * 16, 16)]
            t = t_ref[pl.ds(k * 16, 16)]
            tp = tp_ref[pl.ds(k * 16, 16)]
            ha = (h < N_ITEMS).astype(i32)
            ta = (t < N_ITEMS).astype(i32)
            same_i = 1 - (ha ^ ta)
            flat = h * 32 + same_i * 16 + tp
            gi = base + k * 16 + _iota16()
            val = (gi < N_EDGES).astype(f32)
            idx_ref[k // 8, pl.ds((k % 8) * 16, 16)] = flat
            val_ref[k // 8, pl.ds((k % 8) * 16, 16)] = val

        sds = [pltpu.async_copy(val_ref.at[j], acc.at[idx_ref.at[j]], ssem,
                                add=True) for j in range(4)]
        for sd in sds:
            sd.wait()

    plsc.subcore_barrier()

    @pl.loop(0, 16)
    def _(it):
        ci = sid + it * 16

        @pl.when(ci < nchunk)
        def _():
            pltpu.sync_copy(acc.at[pl.ds(ci * 6400, 6400)], zbuf)
            pltpu.sync_copy(zbuf, out_hbm.at[cid].at[pl.ds(ci * 6400, 6400)])


@jax.jit
def _hist_call(head, tail, etype):
    k = pl.kernel(
        _hist_body,
        out_type=jax.ShapeDtypeStruct((2, NHIST), f32),
        mesh=_MESH,
        compiler_params=_CP,
        scratch_types=[
            pltpu.VMEM((512,), i32),
            pltpu.VMEM((512,), i32),
            pltpu.VMEM((512,), i32),
            pltpu.VMEM((4, 128), i32),
            pltpu.VMEM((4, 128), f32),
            pltpu.VMEM((6400,), f32),
            pltpu.VMEM_SHARED((NHIST,), f32),
            pltpu.SemaphoreType.DMA,
            pltpu.SemaphoreType.DMA,
        ],
    )
    return k(head, tail, etype)


def _scatsum_body(head_hbm, tail_hbm, type_hbm, tab_hbm, out_hbm,
                  h_ref, t_ref, tp_ref, gidx, didx,
                  ent_rows, zrow, acc, gsem, ssem, esem):
    cid = lax.axis_index("c")
    sid = lax.axis_index("s")
    nb = EPAD // (16 * 512)

    @pl.loop(0, 4)
    def _(j):
        _zero_fill2(zrow, 1600)

        @pl.loop(0, 2)
        def _(q):
            pltpu.sync_copy(zrow, acc.at[pl.ds(sid * 3200 + q * 1600, 1600)])

        plsc.subcore_barrier()

        @pl.loop(0, nb)
        def _(b):
            base = (sid * nb + b) * 512
            e1 = pltpu.async_copy(head_hbm.at[pl.ds(base, 512)], h_ref, esem)
            e2 = pltpu.async_copy(tail_hbm.at[pl.ds(base, 512)], t_ref, esem)
            e3 = pltpu.async_copy(type_hbm.at[pl.ds(base, 512)], tp_ref, esem)
            e1.wait()
            e2.wait()
            e3.wait()

            @pl.loop(0, 32)
            def _(k):
                h = h_ref[pl.ds(k * 16, 16)]
                t = t_ref[pl.ds(k * 16, 16)]
                tp = tp_ref[pl.ds(k * 16, 16)]
                ha = (h < N_ITEMS).astype(i32)
                ta = (t < N_ITEMS).astype(i32)
                same_i = 1 - (ha ^ ta)
                gi = base + k * 16 + _iota16()
                valid = (gi < N_EDGES).astype(i32)
                mine = valid * (1 - (same_i ^ cid))
                kr = k // 8
                kc = (k % 8) * 16
                gidx[kr, pl.ds(kc, 16)] = (t + (1 - cid) * (N_ENTITIES
                                                            + tp * N_ENTITIES)) * 4 + j
                didx[kr, pl.ds(kc, 16)] = (mine * h
                                           + (1 - mine) * (N_ENTITIES + _iota16()))

            gds = []
            for q in range(4):
                gds.append(pltpu.async_copy(tab_hbm.at[gidx.at[q]],
                                            ent_rows.at[pl.ds(q * 128, 128)],
                                            gsem))
            for gd in gds:
                gd.wait()

            sds = []
            for q in range(4):
                sds.append(pltpu.async_copy(ent_rows.at[pl.ds(q * 128, 128)],
                                            acc.at[didx.at[q]], ssem, add=True))
            for sd in sds:
                sd.wait()

        plsc.subcore_barrier()

        @pl.loop(0, 2)
        def _(q):
            off = sid * 3200 + q * 1600
            pltpu.sync_copy(acc.at[pl.ds(off, 1600)], zrow)
            pltpu.sync_copy(
                zrow, out_hbm.at[pl.ds((cid * 4 + j) * SROWS + off, 1600)])

        plsc.subcore_barrier()


@jax.jit
def _scatsum_call(head, tail, etype, tab):
    k = pl.kernel(
        _scatsum_body,
        out_type=jax.ShapeDtypeStruct((2 * 4 * SROWS, 16), f32),
        mesh=_MESH,
        compiler_params=_CP,
        scratch_types=[
            pltpu.VMEM((512,), i32),
            pltpu.VMEM((512,), i32),
            pltpu.VMEM((512,), i32),
            pltpu.VMEM((4, 128), i32),
            pltpu.VMEM((4, 128), i32),
            pltpu.VMEM((512, 16), f32),
            pltpu.VMEM((1600, 16), f32),
            pltpu.VMEM_SHARED((SROWS, 16), f32),
            pltpu.SemaphoreType.DMA,
            pltpu.SemaphoreType.DMA,
            pltpu.SemaphoreType.DMA,
        ],
    )
    return k(head, tail, etype, tab)


def _dots_body(rowp_hbm, colp_hbm, u2_hbm, z2_hbm, e_out, d_out,
               row_ref, col_ref, uidx, zidx, didx,
               u_rows, z_rows, dots, e_lin, zd, acc_d, esem, gsem, ssem):
    cid = lax.axis_index("c")
    sid = lax.axis_index("s")

    _zero_fill(zd, 1920)
    pltpu.sync_copy(zd, acc_d.at[pl.ds(sid * 1920, 1920)])
    plsc.subcore_barrier()

    nb = MPAD // (16 * 512)

    @pl.loop(0, nb)
    def _(b):
        base = (sid * nb + b) * 512
        eds = [pltpu.async_copy(rowp_hbm.at[pl.ds(base, 512)], row_ref, esem),
               pltpu.async_copy(colp_hbm.at[pl.ds(base, 512)], col_ref, esem)]
        for ed in eds:
            ed.wait()

        @pl.loop(0, 32)
        def _(k):
            r = row_ref[pl.ds(k * 16, 16)]
            c = col_ref[pl.ds(k * 16, 16)]
            kr = k // 8
            kc = (k % 8) * 16
            uidx[kr, pl.ds(kc, 16)] = cid * N_USERS + r
            zidx[kr, pl.ds(kc, 16)] = cid * N_ITEMS + c
            didx[kr, pl.ds(kc, 16)] = r

        gds = []
        for q in range(4):
            gds.append(pltpu.async_copy(u2_hbm.at[uidx.at[q]],
                                        u_rows.at[pl.ds(q * 128, 128)], gsem))
            gds.append(pltpu.async_copy(z2_hbm.at[zidx.at[q]],
                                        z_rows.at[pl.ds(q * 128, 128)], gsem))
        for gd in gds:
            gd.wait()

        @pl.loop(0, 32)
        def _(g):
            @pl.loop(0, 16)
            def _(r16):
                rr = g * 16 + r16
                part = (u_rows[rr, pl.ds(0, 16)] * z_rows[rr, pl.ds(0, 16)]
                        + u_rows[rr, pl.ds(16, 16)] * z_rows[rr, pl.ds(16, 16)]
                        + u_rows[rr, pl.ds(32, 16)] * z_rows[rr, pl.ds(32, 16)]
                        + u_rows[rr, pl.ds(48, 16)] * z_rows[rr, pl.ds(48, 16)])
                dots[pl.ds(r16 * 16, 16)] = part

            dv = plsc.load_gather(dots, [_iota16() * 16])
            for cc in range(1, 16):
                dv = dv + plsc.load_gather(dots, [_iota16() * 16 + cc])
            sig = 1.0 / (1.0 + jnp.exp(-dv))
            ev = jnp.exp(sig)
            gi = base + g * 16 + _iota16()
            ev = ev * (gi < N_INTER).astype(f32)
            e_lin[pl.ds(g * 16, 16)] = ev

        pltpu.sync_copy(e_lin, e_out.at[cid].at[pl.ds(base, 512)])

        sds = [pltpu.async_copy(e_lin.at[pl.ds(q * 128, 128)],
                                acc_d.at[didx.at[q]], ssem, add=True)
               for q in range(4)]
        for sd in sds:
            sd.wait()

    plsc.subcore_barrier()
    pltpu.sync_copy(acc_d.at[pl.ds(sid * 1920, 1920)], zd)
    pltpu.sync_copy(zd, d_out.at[cid].at[pl.ds(sid * 1920, 1920)])


@jax.jit
def _dots_call(rowp, colp, u2, z2):
    k = pl.kernel(
        _dots_body,
        out_type=(jax.ShapeDtypeStruct((2, MPAD), f32),
                  jax.ShapeDtypeStruct((2, UROWS), f32)),
        mesh=_MESH,
        compiler_params=_CP,
        scratch_types=[
            pltpu.VMEM((512,), i32),
            pltpu.VMEM((512,), i32),
            pltpu.VMEM((4, 128), i32),
            pltpu.VMEM((4, 128), i32),
            pltpu.VMEM((4, 128), i32),
            pltpu.VMEM((512, 64), f32),
            pltpu.VMEM((512, 64), f32),
            pltpu.VMEM((256,), f32),
            pltpu.VMEM((512,), f32),
            pltpu.VMEM((1920,), f32),
            pltpu.VMEM_SHARED((UROWS,), f32),
            pltpu.SemaphoreType.DMA,
            pltpu.SemaphoreType.DMA,
            pltpu.SemaphoreType.DMA,
        ],
    )
    return k(rowp, colp, u2, z2)


def _uacc_body(rowp_hbm, colp_hbm, e2_hbm, dflat_hbm, v2h_hbm,
               usum_out, mask_out,
               row_ref, col_ref, vidx, dpi, dci,
               ep_lin, ec_lin, dp_lin, dc_lin, s_all, mask_lin,
               item_rows, zu, acc_u, esem, gsem, vsem, ssem):
    cid = lax.axis_index("c")
    sid = lax.axis_index("s")

    nb = MPAD // (16 * 512)
    cidf = cid.astype(f32)

    @pl.loop(0, 2)
    def _(jj):
        _zero_fill2(zu, 384, 32)

        @pl.loop(0, 5)
        def _(q):
            pltpu.sync_copy(zu, acc_u.at[pl.ds(sid * 1920 + q * 384, 384)])

        plsc.subcore_barrier()

        @pl.loop(0, nb)
        def _(b):
            base = (sid * nb + b) * 512
            sbase = b * 512
            pltpu.sync_copy(rowp_hbm.at[pl.ds(base, 512)], row_ref)
            pltpu.sync_copy(colp_hbm.at[pl.ds(base, 512)], col_ref)

            @pl.when(jj == 0)
            def _():
                eds = [pltpu.async_copy(e2_hbm.at[0].at[pl.ds(base, 512)],
                                        ep_lin, esem),
                       pltpu.async_copy(e2_hbm.at[1].at[pl.ds(base, 512)],
                                        ec_lin, esem)]
                for ed in eds:
                    ed.wait()

            @pl.loop(0, 32)
            def _(k):
                r = row_ref[pl.ds(k * 16, 16)]
                c = col_ref[pl.ds(k * 16, 16)]
                kr = k // 8
                kc = (k % 8) * 16
                vidx[kr, pl.ds(kc, 16)] = (cid * N_ITEMS + c) * 2 + jj
                dpi[kr, pl.ds(kc, 16)] = r
                dci[kr, pl.ds(kc, 16)] = UROWS + r

            @pl.when(jj == 0)
            def _():
                gds = []
                for q in range(4):
                    gds.append(pltpu.async_copy(dflat_hbm.at[dpi.at[q]],
                                                dp_lin.at[pl.ds(q * 128, 128)],
                                                gsem))
                    gds.append(pltpu.async_copy(dflat_hbm.at[dci.at[q]],
                                                dc_lin.at[pl.ds(q * 128, 128)],
                                                gsem))
                for gd in gds:
                    gd.wait()

            vds = [pltpu.async_copy(v2h_hbm.at[vidx.at[q]],
                                    item_rows.at[pl.ds(q * 128, 128)], vsem)
                   for q in range(4)]
            for vd in vds:
                vd.wait()

            @pl.when(jj == 0)
            def _():
                @pl.loop(0, 32)
                def _(g):
                    sl = pl.ds(g * 16, 16)
                    p = ep_lin[sl] / dp_lin[sl]
                    pcf = ec_lin[sl] / dc_lin[sl]
                    sigp = 1.0 / (1.0 + jnp.exp(-p))
                    sigc = 1.0 / (1.0 + jnp.exp(-pcf))
                    m = (jnp.abs(sigp - sigc) < GAMMA).astype(f32)
                    gi = base + g * 16 + _iota16()
                    validf = (gi < N_INTER).astype(f32)
                    s = (p * (1.0 - cidf) + pcf * cidf) * m * validf
                    s_all[pl.ds(sbase + g * 16, 16)] = s
                    mask_lin[sl] = m.astype(i32)

            @pl.loop(0, 512)
            def _(r):
                sv = plsc.load_gather(s_all, [_iota16() * 0 + (sbase + r)])
                item_rows[r, pl.ds(0, 16)] = item_rows[r, pl.ds(0, 16)] * sv
                item_rows[r, pl.ds(16, 16)] = item_rows[r, pl.ds(16, 16)] * sv

            sds = [pltpu.async_copy(item_rows.at[pl.ds(q * 128, 128)],
                                    acc_u.at[dpi.at[q]], ssem, add=True)
                   for q in range(4)]
            for sd in sds:
                sd.wait()

            @pl.when(cid + jj == 0)
            def _():
                pltpu.sync_copy(mask_lin, mask_out.at[pl.ds(base, 512)])

        plsc.subcore_barrier()

        @pl.loop(0, 5)
        def _(q):
            off = sid * 1920 + q * 384
            pltpu.sync_copy(acc_u.at[pl.ds(off, 384)], zu)
            pltpu.sync_copy(
                zu, usum_out.at[pl.ds((cid * 2 + jj) * UROWS + off, 384)])

        plsc.subcore_barrier()


@jax.jit
def _uacc_call(rowp, colp, e2, dflat, v2h):
    k = pl.kernel(
        _uacc_body,
        out_type=(jax.ShapeDtypeStruct((2 * 2 * UROWS, 32), f32),
                  jax.ShapeDtypeStruct((MPAD,), i32)),
        mesh=_MESH,
        compiler_params=_CP,
        scratch_types=[
            pltpu.VMEM((512,), i32),
            pltpu.VMEM((512,), i32),
            pltpu.VMEM((4, 128), i32),
            pltpu.VMEM((4, 128), i32),
            pltpu.VMEM((4, 128), i32),
            pltpu.VMEM((512,), f32),
            pltpu.VMEM((512,), f32),
            pltpu.VMEM((512,), f32),
            pltpu.VMEM((512,), f32),
            pltpu.VMEM((MPAD // 16,), f32),
            pltpu.VMEM((512,), i32),
            pltpu.VMEM((512, 32), f32),
            pltpu.VMEM((384, 32), f32),
            pltpu.VMEM_SHARED((UROWS, 32), f32),
            pltpu.SemaphoreType.DMA,
            pltpu.SemaphoreType.DMA,
            pltpu.SemaphoreType.DMA,
            pltpu.SemaphoreType.DMA,
        ],
    )
    return k(rowp, colp, e2, dflat, v2h)


def _iagg_body(rowp_hbm, colp_hbm, ucfh_hbm, isum_out, icnt_out,
               row_ref, col_ref, uidx, didx, cval,
               u_rows, zi, zc, acc_i, acc_c, esem, gsem, ssem, csem):
    cid = lax.axis_index("c")
    sid = lax.axis_index("s")

    _zero_fill(zc, 1280)
    pltpu.sync_copy(zc, acc_c.at[pl.ds(sid * 1280, 1280)])

    wid = sid * 2 + cid
    nb = MPAD // (32 * 512)

    @pl.loop(0, 2)
    def _(jj):
        _zero_fill2(zi, 320, 32)

        @pl.loop(0, 4)
        def _(q):
            pltpu.sync_copy(zi, acc_i.at[pl.ds(sid * 1280 + q * 320, 320)])

        plsc.subcore_barrier()

        @pl.loop(0, nb)
        def _(b):
            base = (wid * nb + b) * 512
            eds = [pltpu.async_copy(rowp_hbm.at[pl.ds(base, 512)], row_ref, esem),
                   pltpu.async_copy(colp_hbm.at[pl.ds(base, 512)], col_ref, esem)]
            for ed in eds:
                ed.wait()

            @pl.loop(0, 32)
            def _(k):
                r = row_ref[pl.ds(k * 16, 16)]
                c = col_ref[pl.ds(k * 16, 16)]
                gi = base + k * 16 + _iota16()
                valid = (gi < N_INTER).astype(i32)
                dst = valid * c + (1 - valid) * (N_ITEMS + _iota16())
                kr = k // 8
                kc = (k % 8) * 16
                uidx[kr, pl.ds(kc, 16)] = r * 2 + jj
                didx[kr, pl.ds(kc, 16)] = dst
                cval[kr, pl.ds(kc, 16)] = valid.astype(f32)

            gds = [pltpu.async_copy(ucfh_hbm.at[uidx.at[q]],
                                    u_rows.at[pl.ds(q * 128, 128)], gsem)
                   for q in range(4)]
            for gd in gds:
                gd.wait()

            sds = [pltpu.async_copy(u_rows.at[pl.ds(q * 128, 128)],
                                    acc_i.at[didx.at[q]], ssem, add=True)
                   for q in range(4)]

            @pl.when(jj == 0)
            def _():
                cds = [pltpu.async_copy(cval.at[q], acc_c.at[didx.at[q]],
                                        csem, add=True) for q in range(4)]
                for cd in cds:
                    cd.wait()

            for sd in sds:
                sd.wait()

        plsc.subcore_barrier()

        @pl.loop(0, 4)
        def _(q):
            off = sid * 1280 + q * 320
            pltpu.sync_copy(acc_i.at[pl.ds(off, 320)], zi)
            pltpu.sync_copy(
                zi, isum_out.at[pl.ds((cid * 2 + jj) * IROWS + off, 320)])

        plsc.subcore_barrier()

    pltpu.sync_copy(acc_c.at[pl.ds(sid * 1280, 1280)], zc)
    pltpu.sync_copy(zc, icnt_out.at[cid].at[pl.ds(sid * 1280, 1280)])


@jax.jit
def _iagg_call(rowp, colp, ucfh):
    k = pl.kernel(
        _iagg_body,
        out_type=(jax.ShapeDtypeStruct((2 * 2 * IROWS, 32), f32),
                  jax.ShapeDtypeStruct((2, IROWS), f32)),
        mesh=_MESH,
        compiler_params=_CP,
        scratch_types=[
            pltpu.VMEM((512,), i32),
            pltpu.VMEM((512,), i32),
            pltpu.VMEM((4, 128), i32),
            pltpu.VMEM((4, 128), i32),
            pltpu.VMEM((4, 128), f32),
            pltpu.VMEM((512, 32), f32),
            pltpu.VMEM((320, 32), f32),
            pltpu.VMEM((1280,), f32),
            pltpu.VMEM_SHARED((IROWS, 32), f32),
            pltpu.VMEM_SHARED((IROWS,), f32),
            pltpu.SemaphoreType.DMA,
            pltpu.SemaphoreType.DMA,
            pltpu.SemaphoreType.DMA,
            pltpu.SemaphoreType.DMA,
        ],
    )
    return k(rowp, colp, ucfh)


def kernel(entity_emb, user_emb, user_emb_cf, item_emb_cf, relation_weight,
           W1_w, W1_b, W2_w, W2_b, edge_index, edge_type, interact_mat):
    epad = EPAD - N_EDGES
    pad_h = (jnp.arange(epad, dtype=i32) * 37) % N_ENTITIES
    head = jnp.concatenate([edge_index[0], pad_h])
    tail = jnp.concatenate([edge_index[1], pad_h])
    etype = jnp.pad(edge_type, (0, epad))

    mpad = MPAD - N_INTER
    pad_r = (jnp.arange(mpad, dtype=i32) * 53) % N_USERS
    pad_c = (jnp.arange(mpad, dtype=i32) * 41) % N_ITEMS
    rowp = jnp.concatenate([interact_mat[:, 0], pad_r])
    colp = jnp.concatenate([interact_mat[:, 1], pad_c])

    tab = jnp.concatenate(
        [entity_emb[None], relation_weight[:, None, :] * entity_emb[None]],
        axis=0).reshape(-1, 16)

    hist2 = _hist_call(head, tail, etype)
    s4 = _scatsum_call(head, tail, etype, tab).reshape(2, 4, SROWS, 16)

    hist = (hist2[0] + hist2[1]).reshape(N_ENTITIES, 2, 16)
    cnt_cross = hist[:, 0, :].sum(1)
    cnt_same = hist[:, 1, :].sum(1)
    sum_same_erel = hist[:, 1, :] @ relation_weight
    rel_sum = (hist[:, 0, :] + hist[:, 1, :]) @ relation_weight
    rel_ = rel_sum / jnp.clip(cnt_cross + cnt_same, 1.0, None)[:, None]

    s_all = s4[:, :, :N_ENTITIES, :].transpose(0, 2, 1, 3).reshape(2, N_ENTITIES, DIM)
    S1 = s_all[0]
    S2 = s_all[1] + sum_same_erel
    agg1 = jax.nn.leaky_relu(S1 / jnp.clip(cnt_cross, 1.0, None)[:, None] @ W1_w.T + W1_b, 0.01) / 2.0
    agg2 = jax.nn.leaky_relu(S2 / jnp.clip(cnt_same, 1.0, None)[:, None] @ W2_w.T + W2_b, 0.01) / 2.0
    entity_agg = agg1 + agg2

    item_emb_kg = entity_emb[:N_ITEMS]
    z = rel_[:N_ITEMS] * item_emb_kg
    v2h = jnp.concatenate([item_emb_kg, item_emb_cf]).reshape(2 * 2 * N_ITEMS, 32)
    z2 = jnp.concatenate([z, item_emb_cf])

    u = user_emb
    ucf = user_emb_cf
    mask_pad = None
    for i in range(MAX_ITER):
        u2 = jnp.concatenate([u, ucf])
        e2, d2 = _dots_call(rowp, colp, u2, z2)
        usum, mask_pad = _uacc_call(rowp, colp, e2, d2.reshape(-1), v2h)
        usum = usum.reshape(2, 2, UROWS, 32)
        u = jnp.concatenate([usum[0, 0, :N_USERS], usum[0, 1, :N_USERS]], axis=1)
        ucf = jnp.concatenate([usum[1, 0, :N_USERS], usum[1, 1, :N_USERS]], axis=1)
        if i < MAX_ITER - 1:
            u = u / jnp.clip(jnp.linalg.norm(u, axis=1, keepdims=True), 1e-12, None)
            ucf = ucf / jnp.clip(jnp.linalg.norm(ucf, axis=1, keepdims=True), 1e-12, None)

    ucfh = user_emb_cf.reshape(2 * N_USERS, 32)
    isum, icnt = _iagg_call(rowp, colp, ucfh)
    isum = isum.reshape(2, 2, IROWS, 32)
    ic = jnp.clip(icnt[0, :N_ITEMS] + icnt[1, :N_ITEMS], 1.0, None)[:, None]
    item_sum = jnp.concatenate([isum[0, 0, :N_ITEMS] + isum[1, 0, :N_ITEMS],
                                isum[0, 1, :N_ITEMS] + isum[1, 1, :N_ITEMS]], axis=1)
    item_agg = item_sum / ic

    return (entity_agg, u, ucf, item_agg, mask_pad[:N_INTER])
